# Initial kernel scaffold; baseline (speedup 1.0000x reference)
import jax
import jax.numpy as jnp
from jax.experimental import pallas as pl


def kernel(x, edge_index, W1, b1, W2, b2):
    raise NotImplementedError("write your pallas kernel here")


# speedup vs baseline: 12.3907x; 12.3907x over previous
import functools

import jax
import jax.numpy as jnp
from jax import lax
from jax.experimental import pallas as pl
from jax.experimental.pallas import tpu as pltpu
from jax.experimental.pallas import tpu_sc as plsc

N = 10000
NPAD = 10240
E = 320000
IN_CH = 128
HID_CH = 128
OUT_CH = 64
LANES = 16

NC, NS = 2, 16
NW = NC * NS
EPW = E // NW
C = 40
NCH = EPW // C
RPT = NPAD // NS

BN = 256
GRID = NPAD // BN


def _sc_mesh():
    return plsc.VectorSubcoreMesh(
        core_axis_name="c", subcore_axis_name="s", num_cores=NC, num_subcores=NS
    )


DW = 128


@functools.partial(
    pl.kernel,
    out_type=jax.ShapeDtypeStruct((NC, NPAD, DW), jnp.float32),
    mesh=_sc_mesh(),
    scratch_types=[
        pltpu.VMEM((2, C), jnp.int32),
        pltpu.VMEM((C, DW), jnp.float32),
        pltpu.VMEM((8, DW), jnp.float32),
        pltpu.VMEM_SHARED((NPAD, DW), jnp.float32),
    ]
    + [pltpu.SemaphoreType.DMA] * 2,
)
def _deg_kernel(dst_hbm, out_hbm, dst_v, ones_v, zb, hist, di0, di1):
    cid = lax.axis_index("c")
    sid = lax.axis_index("s")
    wid = cid * NS + sid
    s_di = (di0, di1)
    one = jnp.ones((LANES,), jnp.float32)
    zero = jnp.zeros((LANES,), jnp.float32)
    for r in range(8):
        for l in range(DW // LANES):
            zb[r, pl.ds(l * LANES, LANES)] = zero
    for r in range(C):
        for l in range(DW // LANES):
            ones_v[r, pl.ds(l * LANES, LANES)] = one
    row0 = sid * RPT

    def zloop(k, carry):
        pltpu.sync_copy(zb, hist.at[pl.ds(row0 + k * 8, 8)])
        return carry

    lax.fori_loop(0, RPT // 8, zloop, 0)

    def fetch_idx(j, b):
        off = pl.multiple_of(wid * EPW + j * C, 8)
        pltpu.async_copy(dst_hbm.at[pl.ds(off, C)], dst_v.at[b], s_di[b])

    def wait_idx(b):
        pltpu.make_async_copy(dst_hbm.at[pl.ds(0, C)], dst_v.at[b], s_di[b]).wait()

    def scatter(b):
        pltpu.sync_copy(ones_v, hist.at[dst_v.at[b]], add=True)

    plsc.subcore_barrier()
    fetch_idx(0, 0)
    fetch_idx(1, 1)

    def sloop(g, carry):
        for b in range(2):
            j = g * 2 + b
            wait_idx(b)
            scatter(b)
            fetch_idx(j + 2, b)
        return carry

    lax.fori_loop(0, (NCH - 2) // 2, sloop, 0)
    wait_idx(0)
    scatter(0)
    wait_idx(1)
    scatter(1)
    plsc.subcore_barrier()
    pltpu.sync_copy(hist.at[pl.ds(row0, RPT)], out_hbm.at[cid, pl.ds(row0, RPT)])


D = 128


@functools.partial(
    pl.kernel,
    out_type=jax.ShapeDtypeStruct((NC, NPAD, D), jnp.float32),
    mesh=_sc_mesh(),
    scratch_types=[
        pltpu.VMEM((2, C), jnp.int32),
        pltpu.VMEM((2, C), jnp.int32),
        pltpu.VMEM((2, C, D), jnp.float32),
        pltpu.VMEM((8, D), jnp.float32),
        pltpu.VMEM_SHARED((NPAD, D), jnp.float32),
    ]
    + [pltpu.SemaphoreType.DMA] * 6,
)
def _scatter(hp_hbm, src_hbm, dst_hbm, out_hbm, src_v, dst_v, rows_v, zb, acc,
             gs0, gs1, si0, si1, di0, di1):
    cid = lax.axis_index("c")
    sid = lax.axis_index("s")
    wid = cid * NS + sid
    s_g = (gs0, gs1)
    s_si = (si0, si1)
    s_di = (di0, di1)
    zero = jnp.zeros((LANES,), jnp.float32)
    for r in range(8):
        for l in range(D // LANES):
            zb[r, pl.ds(l * LANES, LANES)] = zero
    row0 = sid * RPT

    def zloop(k, carry):
        pltpu.sync_copy(zb, acc.at[pl.ds(row0 + k * 8, 8)])
        return carry

    lax.fori_loop(0, RPT // 8, zloop, 0)

    def fetch_idx(j, b):
        off = pl.multiple_of(wid * EPW + j * C, 8)
        pltpu.async_copy(src_hbm.at[pl.ds(off, C)], src_v.at[b], s_si[b])
        pltpu.async_copy(dst_hbm.at[pl.ds(off, C)], dst_v.at[b], s_di[b])

    def wait_idx(b):
        pltpu.make_async_copy(src_hbm.at[pl.ds(0, C)], src_v.at[b], s_si[b]).wait()
        pltpu.make_async_copy(dst_hbm.at[pl.ds(0, C)], dst_v.at[b], s_di[b]).wait()

    def start_gather(b):
        pltpu.async_copy(hp_hbm.at[src_v.at[b]], rows_v.at[b], s_g[b])

    def wait_gather(b):
        pltpu.make_async_copy(hp_hbm.at[src_v.at[b]], rows_v.at[b], s_g[b]).wait()

    def scatter(b):
        pltpu.sync_copy(rows_v.at[b], acc.at[dst_v.at[b]], add=True)

    plsc.subcore_barrier()
    fetch_idx(0, 0)
    wait_idx(0)
    fetch_idx(1, 1)
    start_gather(0)

    def cloop(g, carry):
        for b in range(2):
            j = g * 2 + b
            wait_gather(b)
            scatter(b)
            fetch_idx(j + 2, b)
            wait_idx(1 - b)
            start_gather(1 - b)
        return carry

    lax.fori_loop(0, (NCH - 2) // 2, cloop, 0)
    wait_gather(0)
    scatter(0)
    wait_idx(1)
    start_gather(1)
    wait_gather(1)
    scatter(1)

    plsc.subcore_barrier()
    pltpu.sync_copy(acc.at[pl.ds(row0, RPT)], out_hbm.at[cid, pl.ds(row0, RPT)])


def _rows(i):
    return lax.broadcasted_iota(jnp.int32, (BN, 1), 0) + i * BN


def _tc_first(x_pad, W1, degp):
    def body(x_ref, w_ref, degp_ref, hp_ref, dis_ref):
        deg = degp_ref[0, :, 0:1] + degp_ref[1, :, 0:1] + 1.0
        dis = lax.rsqrt(deg)
        h = jnp.dot(x_ref[...], w_ref[...], preferred_element_type=jnp.float32)
        hp = jnp.where(_rows(pl.program_id(0)) < N, h * dis, 0.0)
        hp_ref[...] = hp
        dis_ref[...] = jnp.broadcast_to(dis, (BN, LANES))

    return pl.pallas_call(
        body,
        grid=(GRID,),
        in_specs=[
            pl.BlockSpec((BN, IN_CH), lambda i: (i, 0)),
            pl.BlockSpec((IN_CH, HID_CH), lambda i: (0, 0)),
            pl.BlockSpec((NC, BN, DW), lambda i: (0, i, 0)),
        ],
        out_specs=[
            pl.BlockSpec((BN, HID_CH), lambda i: (i, 0)),
            pl.BlockSpec((BN, LANES), lambda i: (i, 0)),
        ],
        out_shape=[
            jax.ShapeDtypeStruct((NPAD, HID_CH), jnp.float32),
            jax.ShapeDtypeStruct((NPAD, LANES), jnp.float32),
        ],
    )(x_pad, W1, degp)


def _tc_mid(hp1, agg1, dis, b1, W2):
    def body(hp1_ref, agg_ref, dis_ref, b_ref, w_ref, hp2_ref):
        dis_c = dis_ref[:, 0:1]
        s = agg_ref[0] + agg_ref[1] + hp1_ref[...]
        x2 = jnp.maximum(s * dis_c + b_ref[...], 0.0)
        h2 = jnp.dot(x2, w_ref[...], preferred_element_type=jnp.float32)
        hp2 = jnp.where(_rows(pl.program_id(0)) < N, h2 * dis_c, 0.0)
        hp2_ref[...] = jnp.concatenate(
            [hp2, jnp.zeros((BN, HID_CH - OUT_CH), jnp.float32)], axis=1
        )

    return pl.pallas_call(
        body,
        grid=(GRID,),
        in_specs=[
            pl.BlockSpec((BN, HID_CH), lambda i: (i, 0)),
            pl.BlockSpec((NC, BN, HID_CH), lambda i: (0, i, 0)),
            pl.BlockSpec((BN, LANES), lambda i: (i, 0)),
            pl.BlockSpec((1, HID_CH), lambda i: (0, 0)),
            pl.BlockSpec((HID_CH, OUT_CH), lambda i: (0, 0)),
        ],
        out_specs=pl.BlockSpec((BN, HID_CH), lambda i: (i, 0)),
        out_shape=jax.ShapeDtypeStruct((NPAD, HID_CH), jnp.float32),
    )(hp1, agg1, dis, b1, W2)


def _tc_last(hp2, agg2, dis, b2):
    def body(hp2_ref, agg_ref, dis_ref, b_ref, out_ref):
        dis_c = dis_ref[:, 0:1]
        s = agg_ref[0, :, :OUT_CH] + agg_ref[1, :, :OUT_CH] + hp2_ref[:, :OUT_CH]
        out_ref[...] = dis_c * s + b_ref[...]

    return pl.pallas_call(
        body,
        grid=(GRID,),
        in_specs=[
            pl.BlockSpec((BN, HID_CH), lambda i: (i, 0)),
            pl.BlockSpec((NC, BN, HID_CH), lambda i: (0, i, 0)),
            pl.BlockSpec((BN, LANES), lambda i: (i, 0)),
            pl.BlockSpec((1, OUT_CH), lambda i: (0, 0)),
        ],
        out_specs=pl.BlockSpec((BN, OUT_CH), lambda i: (i, 0)),
        out_shape=jax.ShapeDtypeStruct((NPAD, OUT_CH), jnp.float32),
    )(hp2, agg2, dis, b2)


def kernel(x, edge_index, W1, b1, W2, b2):
    src = edge_index[0].astype(jnp.int32)
    dst = edge_index[1].astype(jnp.int32)
    x_pad = jnp.pad(x, ((0, NPAD - N), (0, 0)))
    degp = _deg_kernel(dst)
    hp1, dis = _tc_first(x_pad, W1, degp)
    agg1 = _scatter(hp1, src, dst)
    hp2 = _tc_mid(hp1, agg1, dis, b1.reshape(1, HID_CH), W2)
    agg2 = _scatter(hp2, src, dst)
    out = _tc_last(hp2, agg2, dis, b2.reshape(1, OUT_CH))
    return out[:N]


# speedup vs baseline: 16.2652x; 1.3127x over previous
import functools

import jax
import jax.numpy as jnp
from jax import lax
from jax.experimental import pallas as pl
from jax.experimental.pallas import tpu as pltpu
from jax.experimental.pallas import tpu_sc as plsc

N = 10000
NPAD = 10240
E = 320000
IN_CH = 128
HID_CH = 128
OUT_CH = 64
LANES = 16

NC, NS = 2, 16
NW = NC * NS
EPW = E // NW
C = 80
NCH = EPW // C
RPT = NPAD // NS

BN = 256
GRID = NPAD // BN


def _sc_mesh():
    return plsc.VectorSubcoreMesh(
        core_axis_name="c", subcore_axis_name="s", num_cores=NC, num_subcores=NS
    )


DW = 128


@functools.partial(
    pl.kernel,
    out_type=jax.ShapeDtypeStruct((NC, NPAD, DW), jnp.float32),
    mesh=_sc_mesh(),
    scratch_types=[
        pltpu.VMEM((2, C), jnp.int32),
        pltpu.VMEM((C, DW), jnp.float32),
        pltpu.VMEM((8, DW), jnp.float32),
        pltpu.VMEM_SHARED((NPAD, DW), jnp.float32),
    ]
    + [pltpu.SemaphoreType.DMA] * 2,
)
def _deg_kernel(dst_hbm, ones_hbm, out_hbm, dst_v, ones_v, zb, hist, di0, di1):
    cid = lax.axis_index("c")
    sid = lax.axis_index("s")
    wid = cid * NS + sid
    s_di = (di0, di1)
    zero = jnp.zeros((LANES,), jnp.float32)
    for r in range(8):
        for l in range(DW // LANES):
            zb[r, pl.ds(l * LANES, LANES)] = zero
    pltpu.sync_copy(ones_hbm, ones_v)
    row0 = sid * RPT

    def zloop(k, carry):
        pltpu.sync_copy(zb, hist.at[pl.ds(row0 + k * 8, 8)])
        return carry

    lax.fori_loop(0, RPT // 8, zloop, 0)

    def fetch_idx(j, b):
        off = pl.multiple_of(wid * EPW + j * C, 8)
        pltpu.async_copy(dst_hbm.at[pl.ds(off, C)], dst_v.at[b], s_di[b])

    def wait_idx(b):
        pltpu.make_async_copy(dst_hbm.at[pl.ds(0, C)], dst_v.at[b], s_di[b]).wait()

    def scatter(b):
        pltpu.sync_copy(ones_v, hist.at[dst_v.at[b]], add=True)

    plsc.subcore_barrier()
    fetch_idx(0, 0)
    fetch_idx(1, 1)

    nloop = (NCH - 2) // 2

    def sloop(g, carry):
        for b in range(2):
            j = g * 2 + b
            wait_idx(b)
            scatter(b)
            fetch_idx(j + 2, b)
        return carry

    lax.fori_loop(0, nloop, sloop, 0)
    for j in range(2 * nloop, NCH):
        b = j % 2
        wait_idx(b)
        scatter(b)
        if j + 2 < NCH:
            fetch_idx(j + 2, b)
    plsc.subcore_barrier()
    pltpu.sync_copy(hist.at[pl.ds(row0, RPT)], out_hbm.at[cid, pl.ds(row0, RPT)])


D = 128


@functools.partial(
    pl.kernel,
    out_type=jax.ShapeDtypeStruct((NC, NPAD, D), jnp.float32),
    mesh=_sc_mesh(),
    scratch_types=[
        pltpu.VMEM((2, C), jnp.int32),
        pltpu.VMEM((2, C), jnp.int32),
        pltpu.VMEM((2, C, D), jnp.float32),
        pltpu.VMEM((8, D), jnp.float32),
        pltpu.VMEM_SHARED((NPAD, D), jnp.float32),
    ]
    + [pltpu.SemaphoreType.DMA] * 6,
)
def _scatter(hp_hbm, src_hbm, dst_hbm, out_hbm, src_v, dst_v, rows_v, zb, acc,
             gs0, gs1, si0, si1, di0, di1):
    cid = lax.axis_index("c")
    sid = lax.axis_index("s")
    wid = cid * NS + sid
    s_g = (gs0, gs1)
    s_si = (si0, si1)
    s_di = (di0, di1)
    zero = jnp.zeros((LANES,), jnp.float32)
    for r in range(8):
        for l in range(D // LANES):
            zb[r, pl.ds(l * LANES, LANES)] = zero
    row0 = sid * RPT

    def zloop(k, carry):
        pltpu.sync_copy(zb, acc.at[pl.ds(row0 + k * 8, 8)])
        return carry

    lax.fori_loop(0, RPT // 8, zloop, 0)

    def fetch_idx(j, b):
        off = pl.multiple_of(wid * EPW + j * C, 8)
        pltpu.async_copy(src_hbm.at[pl.ds(off, C)], src_v.at[b], s_si[b])
        pltpu.async_copy(dst_hbm.at[pl.ds(off, C)], dst_v.at[b], s_di[b])

    def wait_idx(b):
        pltpu.make_async_copy(src_hbm.at[pl.ds(0, C)], src_v.at[b], s_si[b]).wait()
        pltpu.make_async_copy(dst_hbm.at[pl.ds(0, C)], dst_v.at[b], s_di[b]).wait()

    def start_gather(b):
        pltpu.async_copy(hp_hbm.at[src_v.at[b]], rows_v.at[b], s_g[b])

    def wait_gather(b):
        pltpu.make_async_copy(hp_hbm.at[src_v.at[b]], rows_v.at[b], s_g[b]).wait()

    def scatter(b):
        pltpu.sync_copy(rows_v.at[b], acc.at[dst_v.at[b]], add=True)

    plsc.subcore_barrier()
    fetch_idx(0, 0)
    wait_idx(0)
    fetch_idx(1, 1)
    start_gather(0)

    nloop = (NCH - 2 - (NCH % 2)) // 2

    def cloop(g, carry):
        for b in range(2):
            j = g * 2 + b
            wait_gather(b)
            scatter(b)
            fetch_idx(j + 2, b)
            wait_idx(1 - b)
            start_gather(1 - b)
        return carry

    lax.fori_loop(0, nloop, cloop, 0)
    for j in range(2 * nloop, NCH):
        b = j % 2
        wait_gather(b)
        scatter(b)
        if j + 2 < NCH:
            fetch_idx(j + 2, b)
        if j + 1 < NCH:
            wait_idx(1 - b)
            start_gather(1 - b)

    plsc.subcore_barrier()
    pltpu.sync_copy(acc.at[pl.ds(row0, RPT)], out_hbm.at[cid, pl.ds(row0, RPT)])


def _rows(i):
    return lax.broadcasted_iota(jnp.int32, (BN, 1), 0) + i * BN


def _tc_first(x_pad, W1, degp):
    def body(x_ref, w_ref, degp_ref, hp_ref, dis_ref):
        deg = degp_ref[0, :, 0:1] + degp_ref[1, :, 0:1] + 1.0
        dis = lax.rsqrt(deg)
        h = jnp.dot(x_ref[...], w_ref[...], preferred_element_type=jnp.float32)
        hp = jnp.where(_rows(pl.program_id(0)) < N, h * dis, 0.0)
        hp_ref[...] = hp
        dis_ref[...] = jnp.broadcast_to(dis, (BN, LANES))

    return pl.pallas_call(
        body,
        grid=(GRID,),
        in_specs=[
            pl.BlockSpec((BN, IN_CH), lambda i: (i, 0)),
            pl.BlockSpec((IN_CH, HID_CH), lambda i: (0, 0)),
            pl.BlockSpec((NC, BN, DW), lambda i: (0, i, 0)),
        ],
        out_specs=[
            pl.BlockSpec((BN, HID_CH), lambda i: (i, 0)),
            pl.BlockSpec((BN, LANES), lambda i: (i, 0)),
        ],
        out_shape=[
            jax.ShapeDtypeStruct((NPAD, HID_CH), jnp.float32),
            jax.ShapeDtypeStruct((NPAD, LANES), jnp.float32),
        ],
    )(x_pad, W1, degp)


def _tc_mid(hp1, agg1, dis, b1, W2):
    def body(hp1_ref, agg_ref, dis_ref, b_ref, w_ref, hp2_ref):
        dis_c = dis_ref[:, 0:1]
        s = agg_ref[0] + agg_ref[1] + hp1_ref[...]
        x2 = jnp.maximum(s * dis_c + b_ref[...], 0.0)
        h2 = jnp.dot(x2, w_ref[...], preferred_element_type=jnp.float32)
        hp2 = jnp.where(_rows(pl.program_id(0)) < N, h2 * dis_c, 0.0)
        hp2_ref[...] = jnp.concatenate(
            [hp2, jnp.zeros((BN, HID_CH - OUT_CH), jnp.float32)], axis=1
        )

    return pl.pallas_call(
        body,
        grid=(GRID,),
        in_specs=[
            pl.BlockSpec((BN, HID_CH), lambda i: (i, 0)),
            pl.BlockSpec((NC, BN, HID_CH), lambda i: (0, i, 0)),
            pl.BlockSpec((BN, LANES), lambda i: (i, 0)),
            pl.BlockSpec((1, HID_CH), lambda i: (0, 0)),
            pl.BlockSpec((HID_CH, OUT_CH), lambda i: (0, 0)),
        ],
        out_specs=pl.BlockSpec((BN, HID_CH), lambda i: (i, 0)),
        out_shape=jax.ShapeDtypeStruct((NPAD, HID_CH), jnp.float32),
    )(hp1, agg1, dis, b1, W2)


def _tc_last(hp2, agg2, dis, b2):
    def body(hp2_ref, agg_ref, dis_ref, b_ref, out_ref):
        dis_c = dis_ref[:, 0:1]
        s = agg_ref[0, :, :OUT_CH] + agg_ref[1, :, :OUT_CH] + hp2_ref[:, :OUT_CH]
        out_ref[...] = dis_c * s + b_ref[...]

    return pl.pallas_call(
        body,
        grid=(GRID,),
        in_specs=[
            pl.BlockSpec((BN, HID_CH), lambda i: (i, 0)),
            pl.BlockSpec((NC, BN, HID_CH), lambda i: (0, i, 0)),
            pl.BlockSpec((BN, LANES), lambda i: (i, 0)),
            pl.BlockSpec((1, OUT_CH), lambda i: (0, 0)),
        ],
        out_specs=pl.BlockSpec((BN, OUT_CH), lambda i: (i, 0)),
        out_shape=jax.ShapeDtypeStruct((NPAD, OUT_CH), jnp.float32),
    )(hp2, agg2, dis, b2)


def kernel(x, edge_index, W1, b1, W2, b2):
    src = edge_index[0].astype(jnp.int32)
    dst = edge_index[1].astype(jnp.int32)
    x_pad = jnp.pad(x, ((0, NPAD - N), (0, 0)))
    degp = _deg_kernel(dst, jnp.ones((C, DW), jnp.float32))
    hp1, dis = _tc_first(x_pad, W1, degp)
    agg1 = _scatter(hp1, src, dst)
    hp2 = _tc_mid(hp1, agg1, dis, b1.reshape(1, HID_CH), W2)
    agg2 = _scatter(hp2, src, dst)
    out = _tc_last(hp2, agg2, dis, b2.reshape(1, OUT_CH))
    return out[:N]


# speedup vs baseline: 19.4867x; 1.1981x over previous
import functools

import jax
import jax.numpy as jnp
from jax import lax
from jax.experimental import pallas as pl
from jax.experimental.pallas import tpu as pltpu
from jax.experimental.pallas import tpu_sc as plsc

N = 10000
NPAD = 10240
E = 320000
IN_CH = 128
HID_CH = 128
OUT_CH = 64
LANES = 16

NC, NS = 2, 16
NW = NC * NS
EPW = E // NW
C = 80
NCH = EPW // C
RPT = NPAD // NS

BN = 256
GRID = NPAD // BN


def _sc_mesh():
    return plsc.VectorSubcoreMesh(
        core_axis_name="c", subcore_axis_name="s", num_cores=NC, num_subcores=NS
    )


DW = 128


@functools.partial(
    pl.kernel,
    out_type=jax.ShapeDtypeStruct((NC, NPAD, DW), jnp.float32),
    mesh=_sc_mesh(),
    scratch_types=[
        pltpu.VMEM((4, C), jnp.int32),
        pltpu.VMEM((C, DW), jnp.float32),
        pltpu.VMEM((8, DW), jnp.float32),
        pltpu.VMEM_SHARED((NPAD, DW), jnp.float32),
    ]
    + [pltpu.SemaphoreType.DMA] * 6,
)
def _deg_kernel(dst_hbm, ones_hbm, out_hbm, dst_v, ones_v, zb, hist, *sems):
    cid = lax.axis_index("c")
    sid = lax.axis_index("s")
    wid = cid * NS + sid
    s_s = sems[:2]
    s_i = sems[2:]
    zero = jnp.zeros((LANES,), jnp.float32)
    for r in range(8):
        for l in range(DW // LANES):
            zb[r, pl.ds(l * LANES, LANES)] = zero
    pltpu.sync_copy(ones_hbm, ones_v)
    row0 = sid * RPT

    def zloop(k, carry):
        pltpu.sync_copy(zb, hist.at[pl.ds(row0 + k * 8, 8)])
        return carry

    lax.fori_loop(0, RPT // 8, zloop, 0)

    def fetch_idx(j, q):
        off = pl.multiple_of(wid * EPW + j * C, 8)
        pltpu.async_copy(dst_hbm.at[pl.ds(off, C)], dst_v.at[q], s_i[q])

    def wait_idx(q):
        pltpu.make_async_copy(dst_hbm.at[pl.ds(0, C)], dst_v.at[q], s_i[q]).wait()

    def ascatter(sb, q):
        pltpu.async_copy(ones_v, hist.at[dst_v.at[q]], s_s[sb], add=True)

    def wscatter(sb, q):
        pltpu.make_async_copy(ones_v, hist.at[dst_v.at[q]], s_s[sb]).wait()

    plsc.subcore_barrier()
    fetch_idx(0, 0)
    fetch_idx(1, 1)
    for j in (0, 1):
        q, sb = j % 4, j % 2
        wait_idx(q)
        ascatter(sb, q)
        fetch_idx(j + 2, (q + 2) % 4)

    def sloop(g, carry):
        j0 = g * 4 + 2
        for k in range(4):
            j = j0 + k
            q, sb = (2 + k) % 4, k % 2
            wait_idx(q)
            wscatter(sb, (q + 2) % 4)
            ascatter(sb, q)
            fetch_idx(j + 2, (q + 2) % 4)
        return carry

    nloop = (NCH - 5) // 4
    lax.fori_loop(0, nloop, sloop, 0)
    for j in range(2 + nloop * 4, NCH):
        q, sb = j % 4, j % 2
        wait_idx(q)
        wscatter(sb, (q + 2) % 4)
        ascatter(sb, q)
        if j + 2 < NCH:
            fetch_idx(j + 2, (q + 2) % 4)
    for j in range(NCH - 2, NCH):
        wscatter(j % 2, j % 4)
    plsc.subcore_barrier()
    pltpu.sync_copy(hist.at[pl.ds(row0, RPT)], out_hbm.at[cid, pl.ds(row0, RPT)])


D = 128


NR = 3
NI = 8


@functools.partial(
    pl.kernel,
    out_type=jax.ShapeDtypeStruct((NC, NPAD, D), jnp.float32),
    mesh=_sc_mesh(),
    scratch_types=[
        pltpu.VMEM((NI, C), jnp.int32),
        pltpu.VMEM((NI, C), jnp.int32),
        pltpu.VMEM((NR, C, D), jnp.float32),
        pltpu.VMEM((8, D), jnp.float32),
        pltpu.VMEM_SHARED((NPAD, D), jnp.float32),
    ]
    + [pltpu.SemaphoreType.DMA] * (2 * NR + NI),
)
def _scatter(hp_hbm, src_hbm, dst_hbm, out_hbm, src_v, dst_v, rows_v, zb, acc,
             *sems):
    cid = lax.axis_index("c")
    sid = lax.axis_index("s")
    wid = cid * NS + sid
    s_g = sems[:NR]
    s_s = sems[NR:2 * NR]
    s_i = sems[2 * NR:]
    zero = jnp.zeros((LANES,), jnp.float32)
    for r in range(8):
        for l in range(D // LANES):
            zb[r, pl.ds(l * LANES, LANES)] = zero
    row0 = sid * RPT

    def zloop(k, carry):
        pltpu.sync_copy(zb, acc.at[pl.ds(row0 + k * 8, 8)])
        return carry

    lax.fori_loop(0, RPT // 8, zloop, 0)

    def fetch_idx(j, q):
        off = pl.multiple_of(wid * EPW + j * C, 8)
        pltpu.async_copy(src_hbm.at[pl.ds(off, C)], src_v.at[q], s_i[q])
        pltpu.async_copy(dst_hbm.at[pl.ds(off, C)], dst_v.at[q], s_i[q])

    def wait_idx(q):
        pltpu.make_async_copy(src_hbm.at[pl.ds(0, C)], src_v.at[q], s_i[q]).wait()
        pltpu.make_async_copy(dst_hbm.at[pl.ds(0, C)], dst_v.at[q], s_i[q]).wait()

    def start_gather(b, q):
        pltpu.async_copy(hp_hbm.at[src_v.at[q]], rows_v.at[b], s_g[b])

    def wait_gather(b, q):
        pltpu.make_async_copy(hp_hbm.at[src_v.at[q]], rows_v.at[b], s_g[b]).wait()

    def ascatter(b, q):
        pltpu.async_copy(rows_v.at[b], acc.at[dst_v.at[q]], s_s[b], add=True)

    def wscatter(b, q):
        pltpu.make_async_copy(rows_v.at[b], acc.at[dst_v.at[q]], s_s[b]).wait()

    plsc.subcore_barrier()
    fetch_idx(0, 0)
    fetch_idx(1, 1)
    fetch_idx(2, 2)
    wait_idx(0)
    start_gather(0, 0)
    for j in (0, 1):
        b, q = j % NR, j % NI
        wait_gather(b, q)
        ascatter(b, q)
        fetch_idx(j + 3, (q + 3) % NI)
        wait_idx((q + 1) % NI)
        start_gather((b + 1) % NR, (q + 1) % NI)

    UNROLL = 24

    def cloop(g, carry):
        j0 = g * UNROLL + 2
        for k in range(UNROLL):
            j = j0 + k
            b, q = (2 + k) % NR, (2 + k) % NI
            wait_gather(b, q)
            ascatter(b, q)
            fetch_idx(j + 3, (q + 3) % NI)
            wait_idx((q + 1) % NI)
            wscatter((b + 1) % NR, (q + 6) % NI)
            start_gather((b + 1) % NR, (q + 1) % NI)
        return carry

    nloop = (NCH - 5) // UNROLL
    lax.fori_loop(0, nloop, cloop, 0)
    for j in range(2 + nloop * UNROLL, NCH):
        b, q = j % NR, j % NI
        wait_gather(b, q)
        ascatter(b, q)
        if j + 3 < NCH:
            fetch_idx(j + 3, (q + 3) % NI)
        if j + 1 < NCH:
            wait_idx((q + 1) % NI)
            if j >= 2:
                wscatter((b + 1) % NR, (q + 6) % NI)
            start_gather((b + 1) % NR, (q + 1) % NI)
    for j in range(NCH - 3, NCH):
        wscatter(j % NR, j % NI)

    plsc.subcore_barrier()
    pltpu.sync_copy(acc.at[pl.ds(row0, RPT)], out_hbm.at[cid, pl.ds(row0, RPT)])


def _rows(i):
    return lax.broadcasted_iota(jnp.int32, (BN, 1), 0) + i * BN


def _tc_first(x_pad, W1, degp):
    def body(x_ref, w_ref, degp_ref, hp_ref, dis_ref):
        deg = degp_ref[0, :, 0:1] + degp_ref[1, :, 0:1] + 1.0
        dis = lax.rsqrt(deg)
        h = jnp.dot(x_ref[...], w_ref[...], preferred_element_type=jnp.float32)
        hp = jnp.where(_rows(pl.program_id(0)) < N, h * dis, 0.0)
        hp_ref[...] = hp
        dis_ref[...] = jnp.broadcast_to(dis, (BN, LANES))

    return pl.pallas_call(
        body,
        grid=(GRID,),
        in_specs=[
            pl.BlockSpec((BN, IN_CH), lambda i: (i, 0)),
            pl.BlockSpec((IN_CH, HID_CH), lambda i: (0, 0)),
            pl.BlockSpec((NC, BN, DW), lambda i: (0, i, 0)),
        ],
        out_specs=[
            pl.BlockSpec((BN, HID_CH), lambda i: (i, 0)),
            pl.BlockSpec((BN, LANES), lambda i: (i, 0)),
        ],
        out_shape=[
            jax.ShapeDtypeStruct((NPAD, HID_CH), jnp.float32),
            jax.ShapeDtypeStruct((NPAD, LANES), jnp.float32),
        ],
    )(x_pad, W1, degp)


def _tc_mid(hp1, agg1, dis, b1, W2):
    def body(hp1_ref, agg_ref, dis_ref, b_ref, w_ref, hp2_ref):
        dis_c = dis_ref[:, 0:1]
        s = agg_ref[0] + agg_ref[1] + hp1_ref[...]
        x2 = jnp.maximum(s * dis_c + b_ref[...], 0.0)
        h2 = jnp.dot(x2, w_ref[...], preferred_element_type=jnp.float32)
        hp2 = jnp.where(_rows(pl.program_id(0)) < N, h2 * dis_c, 0.0)
        hp2_ref[...] = jnp.concatenate(
            [hp2, jnp.zeros((BN, HID_CH - OUT_CH), jnp.float32)], axis=1
        )

    return pl.pallas_call(
        body,
        grid=(GRID,),
        in_specs=[
            pl.BlockSpec((BN, HID_CH), lambda i: (i, 0)),
            pl.BlockSpec((NC, BN, HID_CH), lambda i: (0, i, 0)),
            pl.BlockSpec((BN, LANES), lambda i: (i, 0)),
            pl.BlockSpec((1, HID_CH), lambda i: (0, 0)),
            pl.BlockSpec((HID_CH, OUT_CH), lambda i: (0, 0)),
        ],
        out_specs=pl.BlockSpec((BN, HID_CH), lambda i: (i, 0)),
        out_shape=jax.ShapeDtypeStruct((NPAD, HID_CH), jnp.float32),
    )(hp1, agg1, dis, b1, W2)


def _tc_last(hp2, agg2, dis, b2):
    def body(hp2_ref, agg_ref, dis_ref, b_ref, out_ref):
        dis_c = dis_ref[:, 0:1]
        s = agg_ref[0, :, :OUT_CH] + agg_ref[1, :, :OUT_CH] + hp2_ref[:, :OUT_CH]
        out_ref[...] = dis_c * s + b_ref[...]

    return pl.pallas_call(
        body,
        grid=(GRID,),
        in_specs=[
            pl.BlockSpec((BN, HID_CH), lambda i: (i, 0)),
            pl.BlockSpec((NC, BN, HID_CH), lambda i: (0, i, 0)),
            pl.BlockSpec((BN, LANES), lambda i: (i, 0)),
            pl.BlockSpec((1, OUT_CH), lambda i: (0, 0)),
        ],
        out_specs=pl.BlockSpec((BN, OUT_CH), lambda i: (i, 0)),
        out_shape=jax.ShapeDtypeStruct((NPAD, OUT_CH), jnp.float32),
    )(hp2, agg2, dis, b2)


def kernel(x, edge_index, W1, b1, W2, b2):
    src = edge_index[0].astype(jnp.int32)
    dst = edge_index[1].astype(jnp.int32)
    x_pad = jnp.pad(x, ((0, NPAD - N), (0, 0)))
    degp = _deg_kernel(dst, jnp.ones((C, DW), jnp.float32))
    hp1, dis = _tc_first(x_pad, W1, degp)
    agg1 = _scatter(hp1, src, dst)
    hp2 = _tc_mid(hp1, agg1, dis, b1.reshape(1, HID_CH), W2)
    agg2 = _scatter(hp2, src, dst)
    out = _tc_last(hp2, agg2, dis, b2.reshape(1, OUT_CH))
    return out[:N]


# speedup vs baseline: 21.3059x; 1.0934x over previous
import functools

import jax
import jax.numpy as jnp
from jax import lax
from jax.experimental import pallas as pl
from jax.experimental.pallas import tpu as pltpu
from jax.experimental.pallas import tpu_sc as plsc

N = 10000
NPAD = 10240
E = 320000
IN_CH = 128
HID_CH = 128
OUT_CH = 64
LANES = 16

NC, NS = 2, 16
NW = NC * NS
EPW = E // NW
C = 80
NCH = EPW // C
RPT = NPAD // NS

BN = 256
GRID = NPAD // BN


def _sc_mesh():
    return plsc.VectorSubcoreMesh(
        core_axis_name="c", subcore_axis_name="s", num_cores=NC, num_subcores=NS
    )


DEGW = 16
NGRP = EPW // LANES
KGRP = RPT // LANES


@functools.partial(
    pl.kernel,
    out_type=jax.ShapeDtypeStruct((NC, NPAD, DEGW), jnp.float32),
    mesh=_sc_mesh(),
    compiler_params=pltpu.CompilerParams(needs_layout_passes=False),
    scratch_types=[
        pltpu.VMEM((EPW,), jnp.int32),
        pltpu.VMEM((NPAD,), jnp.float32),
        pltpu.VMEM((NS, RPT), jnp.float32),
        pltpu.VMEM((RPT, DEGW), jnp.float32),
        pltpu.VMEM_SHARED((NS, NS, RPT), jnp.float32),
    ],
)
def _deg_kernel(dst_hbm, out_hbm, dst_v, hist, part_v, deg_v, shared):
    cid = lax.axis_index("c")
    sid = lax.axis_index("s")
    wid = cid * NS + sid
    zero = jnp.zeros((LANES,), jnp.float32)

    def zloop(k, carry):
        hist[pl.ds(k * LANES, LANES)] = zero
        return carry

    lax.fori_loop(0, NPAD // LANES, zloop, 0)
    off = pl.multiple_of(wid * EPW, 8)
    pltpu.sync_copy(dst_hbm.at[pl.ds(off, EPW)], dst_v)

    def hloop(g, carry):
        d = dst_v[pl.ds(g * LANES, LANES)]
        occ, last = plsc.scan_count(d)
        plsc.addupdate_scatter(
            hist, (d,), lax.convert_element_type(occ, jnp.float32), mask=last
        )
        return carry

    lax.fori_loop(0, NGRP, hloop, 0)

    for t in range(NS):
        pltpu.sync_copy(hist.at[pl.ds(t * RPT, RPT)], shared.at[t, sid])
    plsc.subcore_barrier()
    pltpu.sync_copy(shared.at[sid], part_v)
    for k in range(KGRP):
        acc = jnp.zeros((LANES,), jnp.float32)
        for r in range(NS):
            acc = acc + part_v[r, pl.ds(k * LANES, LANES)]
        for i in range(LANES):
            deg_v[k * LANES + i, :] = jnp.take(
                acc, jnp.full((DEGW,), i, jnp.int32)
            )
    row0 = sid * RPT
    pltpu.sync_copy(deg_v, out_hbm.at[cid, pl.ds(row0, RPT)])


D = 128


NR = 3
NI = 8


@functools.partial(
    pl.kernel,
    out_type=jax.ShapeDtypeStruct((NC, NPAD, D), jnp.float32),
    mesh=_sc_mesh(),
    scratch_types=[
        pltpu.VMEM((NI, C), jnp.int32),
        pltpu.VMEM((NI, C), jnp.int32),
        pltpu.VMEM((NR, C, D), jnp.float32),
        pltpu.VMEM((8, D), jnp.float32),
        pltpu.VMEM_SHARED((NPAD, D), jnp.float32),
    ]
    + [pltpu.SemaphoreType.DMA] * (2 * NR + NI),
)
def _scatter(hp_hbm, src_hbm, dst_hbm, out_hbm, src_v, dst_v, rows_v, zb, acc,
             *sems):
    cid = lax.axis_index("c")
    sid = lax.axis_index("s")
    wid = cid * NS + sid
    s_g = sems[:NR]
    s_s = sems[NR:2 * NR]
    s_i = sems[2 * NR:]
    zero = jnp.zeros((LANES,), jnp.float32)
    for r in range(8):
        for l in range(D // LANES):
            zb[r, pl.ds(l * LANES, LANES)] = zero
    row0 = sid * RPT

    def zloop(k, carry):
        pltpu.sync_copy(zb, acc.at[pl.ds(row0 + k * 8, 8)])
        return carry

    lax.fori_loop(0, RPT // 8, zloop, 0)

    def fetch_idx(j, q):
        off = pl.multiple_of(wid * EPW + j * C, 8)
        pltpu.async_copy(src_hbm.at[pl.ds(off, C)], src_v.at[q], s_i[q])
        pltpu.async_copy(dst_hbm.at[pl.ds(off, C)], dst_v.at[q], s_i[q])

    def wait_idx(q):
        pltpu.make_async_copy(src_hbm.at[pl.ds(0, C)], src_v.at[q], s_i[q]).wait()
        pltpu.make_async_copy(dst_hbm.at[pl.ds(0, C)], dst_v.at[q], s_i[q]).wait()

    def start_gather(b, q):
        pltpu.async_copy(hp_hbm.at[src_v.at[q]], rows_v.at[b], s_g[b])

    def wait_gather(b, q):
        pltpu.make_async_copy(hp_hbm.at[src_v.at[q]], rows_v.at[b], s_g[b]).wait()

    def ascatter(b, q):
        pltpu.async_copy(rows_v.at[b], acc.at[dst_v.at[q]], s_s[b], add=True)

    def wscatter(b, q):
        pltpu.make_async_copy(rows_v.at[b], acc.at[dst_v.at[q]], s_s[b]).wait()

    plsc.subcore_barrier()
    fetch_idx(0, 0)
    fetch_idx(1, 1)
    fetch_idx(2, 2)
    wait_idx(0)
    start_gather(0, 0)
    for j in (0, 1):
        b, q = j % NR, j % NI
        wait_gather(b, q)
        ascatter(b, q)
        fetch_idx(j + 3, (q + 3) % NI)
        wait_idx((q + 1) % NI)
        start_gather((b + 1) % NR, (q + 1) % NI)

    UNROLL = 24

    def cloop(g, carry):
        j0 = g * UNROLL + 2
        for k in range(UNROLL):
            j = j0 + k
            b, q = (2 + k) % NR, (2 + k) % NI
            wait_gather(b, q)
            ascatter(b, q)
            fetch_idx(j + 3, (q + 3) % NI)
            wait_idx((q + 1) % NI)
            wscatter((b + 1) % NR, (q + 6) % NI)
            start_gather((b + 1) % NR, (q + 1) % NI)
        return carry

    nloop = (NCH - 5) // UNROLL
    lax.fori_loop(0, nloop, cloop, 0)
    for j in range(2 + nloop * UNROLL, NCH):
        b, q = j % NR, j % NI
        wait_gather(b, q)
        ascatter(b, q)
        if j + 3 < NCH:
            fetch_idx(j + 3, (q + 3) % NI)
        if j + 1 < NCH:
            wait_idx((q + 1) % NI)
            if j >= 2:
                wscatter((b + 1) % NR, (q + 6) % NI)
            start_gather((b + 1) % NR, (q + 1) % NI)
    for j in range(NCH - 3, NCH):
        wscatter(j % NR, j % NI)

    plsc.subcore_barrier()
    pltpu.sync_copy(acc.at[pl.ds(row0, RPT)], out_hbm.at[cid, pl.ds(row0, RPT)])


def _rows(i):
    return lax.broadcasted_iota(jnp.int32, (BN, 1), 0) + i * BN


def _tc_matmul1(x_pad, W1):
    def body(x_ref, w_ref, h_ref):
        h_ref[...] = jnp.dot(
            x_ref[...], w_ref[...], preferred_element_type=jnp.float32
        )

    return pl.pallas_call(
        body,
        grid=(GRID,),
        in_specs=[
            pl.BlockSpec((BN, IN_CH), lambda i: (i, 0)),
            pl.BlockSpec((IN_CH, HID_CH), lambda i: (0, 0)),
        ],
        out_specs=pl.BlockSpec((BN, HID_CH), lambda i: (i, 0)),
        out_shape=jax.ShapeDtypeStruct((NPAD, HID_CH), jnp.float32),
    )(x_pad, W1)


def _tc_scale(h1, degp):
    def body(h_ref, degp_ref, hp_ref, dis_ref):
        deg = degp_ref[0, :, 0:1] + degp_ref[1, :, 0:1] + 1.0
        dis = lax.rsqrt(deg)
        hp = jnp.where(_rows(pl.program_id(0)) < N, h_ref[...] * dis, 0.0)
        hp_ref[...] = hp
        dis_ref[...] = jnp.broadcast_to(dis, (BN, LANES))

    return pl.pallas_call(
        body,
        grid=(GRID,),
        in_specs=[
            pl.BlockSpec((BN, HID_CH), lambda i: (i, 0)),
            pl.BlockSpec((NC, BN, DEGW), lambda i: (0, i, 0)),
        ],
        out_specs=[
            pl.BlockSpec((BN, HID_CH), lambda i: (i, 0)),
            pl.BlockSpec((BN, LANES), lambda i: (i, 0)),
        ],
        out_shape=[
            jax.ShapeDtypeStruct((NPAD, HID_CH), jnp.float32),
            jax.ShapeDtypeStruct((NPAD, LANES), jnp.float32),
        ],
    )(h1, degp)


def _tc_mid(hp1, agg1, dis, b1, W2):
    def body(hp1_ref, agg_ref, dis_ref, b_ref, w_ref, hp2_ref):
        dis_c = dis_ref[:, 0:1]
        s = agg_ref[0] + agg_ref[1] + hp1_ref[...]
        x2 = jnp.maximum(s * dis_c + b_ref[...], 0.0)
        h2 = jnp.dot(x2, w_ref[...], preferred_element_type=jnp.float32)
        hp2 = jnp.where(_rows(pl.program_id(0)) < N, h2 * dis_c, 0.0)
        hp2_ref[...] = jnp.concatenate(
            [hp2, jnp.zeros((BN, HID_CH - OUT_CH), jnp.float32)], axis=1
        )

    return pl.pallas_call(
        body,
        grid=(GRID,),
        in_specs=[
            pl.BlockSpec((BN, HID_CH), lambda i: (i, 0)),
            pl.BlockSpec((NC, BN, HID_CH), lambda i: (0, i, 0)),
            pl.BlockSpec((BN, LANES), lambda i: (i, 0)),
            pl.BlockSpec((1, HID_CH), lambda i: (0, 0)),
            pl.BlockSpec((HID_CH, OUT_CH), lambda i: (0, 0)),
        ],
        out_specs=pl.BlockSpec((BN, HID_CH), lambda i: (i, 0)),
        out_shape=jax.ShapeDtypeStruct((NPAD, HID_CH), jnp.float32),
    )(hp1, agg1, dis, b1, W2)


def _tc_last(hp2, agg2, dis, b2):
    def body(hp2_ref, agg_ref, dis_ref, b_ref, out_ref):
        dis_c = dis_ref[:, 0:1]
        s = agg_ref[0, :, :OUT_CH] + agg_ref[1, :, :OUT_CH] + hp2_ref[:, :OUT_CH]
        out_ref[...] = dis_c * s + b_ref[...]

    return pl.pallas_call(
        body,
        grid=(GRID,),
        in_specs=[
            pl.BlockSpec((BN, HID_CH), lambda i: (i, 0)),
            pl.BlockSpec((NC, BN, HID_CH), lambda i: (0, i, 0)),
            pl.BlockSpec((BN, LANES), lambda i: (i, 0)),
            pl.BlockSpec((1, OUT_CH), lambda i: (0, 0)),
        ],
        out_specs=pl.BlockSpec((BN, OUT_CH), lambda i: (i, 0)),
        out_shape=jax.ShapeDtypeStruct((NPAD, OUT_CH), jnp.float32),
    )(hp2, agg2, dis, b2)


def kernel(x, edge_index, W1, b1, W2, b2):
    src = edge_index[0].astype(jnp.int32)
    dst = edge_index[1].astype(jnp.int32)
    x_pad = jnp.pad(x, ((0, NPAD - N), (0, 0)))
    degp = _deg_kernel(dst)
    h1 = _tc_matmul1(x_pad, W1)
    hp1, dis = _tc_scale(h1, degp)
    agg1 = _scatter(hp1, src, dst)
    hp2 = _tc_mid(hp1, agg1, dis, b1.reshape(1, HID_CH), W2)
    agg2 = _scatter(hp2, src, dst)
    out = _tc_last(hp2, agg2, dis, b2.reshape(1, OUT_CH))
    return out[:N]


# speedup vs baseline: 23.0831x; 1.0834x over previous
import functools

import jax
import jax.numpy as jnp
from jax import lax
from jax.experimental import pallas as pl
from jax.experimental.pallas import tpu as pltpu
from jax.experimental.pallas import tpu_sc as plsc

N = 10000
NPAD = 10240
E = 320000
IN_CH = 128
HID_CH = 128
OUT_CH = 64
LANES = 16

NC, NS = 2, 16
NW = NC * NS
EPW = E // NW
C = 40
NCH = EPW // C
RPT = NPAD // NS

BN = 256
GRID = NPAD // BN


def _sc_mesh():
    return plsc.VectorSubcoreMesh(
        core_axis_name="c", subcore_axis_name="s", num_cores=NC, num_subcores=NS
    )


DEGW = 16
NGRP = EPW // LANES
KGRP = RPT // LANES


@functools.partial(
    pl.kernel,
    out_type=jax.ShapeDtypeStruct((NC, NPAD, DEGW), jnp.float32),
    mesh=_sc_mesh(),
    compiler_params=pltpu.CompilerParams(needs_layout_passes=False),
    scratch_types=[
        pltpu.VMEM((EPW,), jnp.int32),
        pltpu.VMEM((NPAD,), jnp.float32),
        pltpu.VMEM((NS, RPT), jnp.float32),
        pltpu.VMEM((RPT, DEGW), jnp.float32),
        pltpu.VMEM_SHARED((NS, NS, RPT), jnp.float32),
    ],
)
def _deg_kernel(dst_hbm, out_hbm, dst_v, hist, part_v, deg_v, shared):
    cid = lax.axis_index("c")
    sid = lax.axis_index("s")
    wid = cid * NS + sid
    zero = jnp.zeros((LANES,), jnp.float32)

    def zloop(k, carry):
        hist[pl.ds(k * LANES, LANES)] = zero
        return carry

    lax.fori_loop(0, NPAD // LANES, zloop, 0)
    off = pl.multiple_of(wid * EPW, 8)
    pltpu.sync_copy(dst_hbm.at[pl.ds(off, EPW)], dst_v)

    def hloop(g, carry):
        d = dst_v[pl.ds(g * LANES, LANES)]
        occ, last = plsc.scan_count(d)
        plsc.addupdate_scatter(
            hist, (d,), lax.convert_element_type(occ, jnp.float32), mask=last
        )
        return carry

    lax.fori_loop(0, NGRP, hloop, 0)

    for t in range(NS):
        pltpu.sync_copy(hist.at[pl.ds(t * RPT, RPT)], shared.at[t, sid])
    plsc.subcore_barrier()
    pltpu.sync_copy(shared.at[sid], part_v)
    for k in range(KGRP):
        acc = jnp.zeros((LANES,), jnp.float32)
        for r in range(NS):
            acc = acc + part_v[r, pl.ds(k * LANES, LANES)]
        for i in range(LANES):
            deg_v[k * LANES + i, :] = jnp.take(
                acc, jnp.full((DEGW,), i, jnp.int32)
            )
    row0 = sid * RPT
    pltpu.sync_copy(deg_v, out_hbm.at[cid, pl.ds(row0, RPT)])


D = 128


NR = 5
NI = 10


@functools.partial(
    pl.kernel,
    out_type=jax.ShapeDtypeStruct((NC, NPAD, D), jnp.float32),
    mesh=_sc_mesh(),
    scratch_types=[
        pltpu.VMEM((NI, C), jnp.int32),
        pltpu.VMEM((NI, C), jnp.int32),
        pltpu.VMEM((NR, C, D), jnp.float32),
        pltpu.VMEM((8, D), jnp.float32),
        pltpu.VMEM_SHARED((NPAD, D), jnp.float32),
    ]
    + [pltpu.SemaphoreType.DMA] * (2 * NR + NI),
)
def _scatter(hp_hbm, src_hbm, dst_hbm, out_hbm, src_v, dst_v, rows_v, zb, acc,
             *sems):
    cid = lax.axis_index("c")
    sid = lax.axis_index("s")
    wid = cid * NS + sid
    s_g = sems[:NR]
    s_s = sems[NR:2 * NR]
    s_i = sems[2 * NR:]
    zero = jnp.zeros((LANES,), jnp.float32)
    for r in range(8):
        for l in range(D // LANES):
            zb[r, pl.ds(l * LANES, LANES)] = zero
    row0 = sid * RPT

    def zloop(k, carry):
        pltpu.sync_copy(zb, acc.at[pl.ds(row0 + k * 8, 8)])
        return carry

    lax.fori_loop(0, RPT // 8, zloop, 0)

    def fetch_idx(j, q):
        off = pl.multiple_of(wid * EPW + j * C, 8)
        pltpu.async_copy(src_hbm.at[pl.ds(off, C)], src_v.at[q], s_i[q])
        pltpu.async_copy(dst_hbm.at[pl.ds(off, C)], dst_v.at[q], s_i[q])

    def wait_idx(q):
        pltpu.make_async_copy(src_hbm.at[pl.ds(0, C)], src_v.at[q], s_i[q]).wait()
        pltpu.make_async_copy(dst_hbm.at[pl.ds(0, C)], dst_v.at[q], s_i[q]).wait()

    def start_gather(b, q):
        pltpu.async_copy(hp_hbm.at[src_v.at[q]], rows_v.at[b], s_g[b])

    def wait_gather(b, q):
        pltpu.make_async_copy(hp_hbm.at[src_v.at[q]], rows_v.at[b], s_g[b]).wait()

    def ascatter(b, q):
        pltpu.async_copy(rows_v.at[b], acc.at[dst_v.at[q]], s_s[b], add=True)

    def wscatter(b, q):
        pltpu.make_async_copy(rows_v.at[b], acc.at[dst_v.at[q]], s_s[b]).wait()

    plsc.subcore_barrier()

    def step(j, b, q, retire, fetch_ok, gather_ok):
        wait_gather(b, q)
        ascatter(b, q)
        if retire:
            wscatter((b + 2) % NR, (q + 7) % NI)
        if fetch_ok:
            fetch_idx(j + 4, (q + 4) % NI)
        if gather_ok:
            wait_idx((q + 2) % NI)
            start_gather((b + 2) % NR, (q + 2) % NI)

    for q in range(4):
        fetch_idx(q, q)
    wait_idx(0)
    start_gather(0, 0)
    wait_idx(1)
    start_gather(1, 1)
    for j in (0, 1, 2):
        step(j, j % NR, j % NI, False, True, True)

    UNROLL = 10

    def cloop(g, carry):
        j0 = g * UNROLL + 3
        for k in range(UNROLL):
            step(j0 + k, (3 + k) % NR, (3 + k) % NI, True, True, True)
        return carry

    nloop = (NCH - 3 - 7) // UNROLL
    lax.fori_loop(0, nloop, cloop, 0)
    for j in range(3 + nloop * UNROLL, NCH):
        step(j, j % NR, j % NI, True, j + 4 < NCH, j + 2 < NCH)
    for j in range(NCH - 3, NCH):
        wscatter(j % NR, j % NI)

    plsc.subcore_barrier()
    pltpu.sync_copy(acc.at[pl.ds(row0, RPT)], out_hbm.at[cid, pl.ds(row0, RPT)])


def _rows(i):
    return lax.broadcasted_iota(jnp.int32, (BN, 1), 0) + i * BN


def _tc_matmul1(x_pad, W1):
    def body(x_ref, w_ref, h_ref):
        h_ref[...] = jnp.dot(
            x_ref[...], w_ref[...], preferred_element_type=jnp.float32
        )

    return pl.pallas_call(
        body,
        grid=(GRID,),
        in_specs=[
            pl.BlockSpec((BN, IN_CH), lambda i: (i, 0)),
            pl.BlockSpec((IN_CH, HID_CH), lambda i: (0, 0)),
        ],
        out_specs=pl.BlockSpec((BN, HID_CH), lambda i: (i, 0)),
        out_shape=jax.ShapeDtypeStruct((NPAD, HID_CH), jnp.float32),
    )(x_pad, W1)


def _tc_scale(h1, degp):
    def body(h_ref, degp_ref, hp_ref, dis_ref):
        deg = degp_ref[0, :, 0:1] + degp_ref[1, :, 0:1] + 1.0
        dis = lax.rsqrt(deg)
        hp = jnp.where(_rows(pl.program_id(0)) < N, h_ref[...] * dis, 0.0)
        hp_ref[...] = hp
        dis_ref[...] = jnp.broadcast_to(dis, (BN, LANES))

    return pl.pallas_call(
        body,
        grid=(GRID,),
        in_specs=[
            pl.BlockSpec((BN, HID_CH), lambda i: (i, 0)),
            pl.BlockSpec((NC, BN, DEGW), lambda i: (0, i, 0)),
        ],
        out_specs=[
            pl.BlockSpec((BN, HID_CH), lambda i: (i, 0)),
            pl.BlockSpec((BN, LANES), lambda i: (i, 0)),
        ],
        out_shape=[
            jax.ShapeDtypeStruct((NPAD, HID_CH), jnp.float32),
            jax.ShapeDtypeStruct((NPAD, LANES), jnp.float32),
        ],
    )(h1, degp)


def _tc_mid(hp1, agg1, dis, b1, W2):
    def body(hp1_ref, agg_ref, dis_ref, b_ref, w_ref, hp2_ref):
        dis_c = dis_ref[:, 0:1]
        s = agg_ref[0] + agg_ref[1] + hp1_ref[...]
        x2 = jnp.maximum(s * dis_c + b_ref[...], 0.0)
        h2 = jnp.dot(x2, w_ref[...], preferred_element_type=jnp.float32)
        hp2 = jnp.where(_rows(pl.program_id(0)) < N, h2 * dis_c, 0.0)
        hp2_ref[...] = jnp.concatenate(
            [hp2, jnp.zeros((BN, HID_CH - OUT_CH), jnp.float32)], axis=1
        )

    return pl.pallas_call(
        body,
        grid=(GRID,),
        in_specs=[
            pl.BlockSpec((BN, HID_CH), lambda i: (i, 0)),
            pl.BlockSpec((NC, BN, HID_CH), lambda i: (0, i, 0)),
            pl.BlockSpec((BN, LANES), lambda i: (i, 0)),
            pl.BlockSpec((1, HID_CH), lambda i: (0, 0)),
            pl.BlockSpec((HID_CH, OUT_CH), lambda i: (0, 0)),
        ],
        out_specs=pl.BlockSpec((BN, HID_CH), lambda i: (i, 0)),
        out_shape=jax.ShapeDtypeStruct((NPAD, HID_CH), jnp.float32),
    )(hp1, agg1, dis, b1, W2)


def _tc_last(hp2, agg2, dis, b2):
    def body(hp2_ref, agg_ref, dis_ref, b_ref, out_ref):
        dis_c = dis_ref[:, 0:1]
        s = agg_ref[0, :, :OUT_CH] + agg_ref[1, :, :OUT_CH] + hp2_ref[:, :OUT_CH]
        out_ref[...] = dis_c * s + b_ref[...]

    return pl.pallas_call(
        body,
        grid=(GRID,),
        in_specs=[
            pl.BlockSpec((BN, HID_CH), lambda i: (i, 0)),
            pl.BlockSpec((NC, BN, HID_CH), lambda i: (0, i, 0)),
            pl.BlockSpec((BN, LANES), lambda i: (i, 0)),
            pl.BlockSpec((1, OUT_CH), lambda i: (0, 0)),
        ],
        out_specs=pl.BlockSpec((BN, OUT_CH), lambda i: (i, 0)),
        out_shape=jax.ShapeDtypeStruct((NPAD, OUT_CH), jnp.float32),
    )(hp2, agg2, dis, b2)


def kernel(x, edge_index, W1, b1, W2, b2):
    src = edge_index[0].astype(jnp.int32)
    dst = edge_index[1].astype(jnp.int32)
    x_pad = jnp.pad(x, ((0, NPAD - N), (0, 0)))
    degp = _deg_kernel(dst)
    h1 = _tc_matmul1(x_pad, W1)
    hp1, dis = _tc_scale(h1, degp)
    agg1 = _scatter(hp1, src, dst)
    hp2 = _tc_mid(hp1, agg1, dis, b1.reshape(1, HID_CH), W2)
    agg2 = _scatter(hp2, src, dst)
    out = _tc_last(hp2, agg2, dis, b2.reshape(1, OUT_CH))
    return out[:N]


# speedup vs baseline: 27.1551x; 1.1764x over previous
import functools

import jax
import jax.numpy as jnp
from jax import lax
from jax.experimental import pallas as pl
from jax.experimental.pallas import tpu as pltpu
from jax.experimental.pallas import tpu_sc as plsc

N = 10000
NPAD = 10240
E = 320000
IN_CH = 128
HID_CH = 128
OUT_CH = 64
LANES = 16

NC, NS = 2, 16
NW = NC * NS
EPW = E // NW
C = 40
NCH = EPW // C
RPT = NPAD // NS

BN = 256
GRID = NPAD // BN


def _sc_mesh():
    return plsc.VectorSubcoreMesh(
        core_axis_name="c", subcore_axis_name="s", num_cores=NC, num_subcores=NS
    )


DEGW = 16
NGRP = EPW // LANES
KGRP = RPT // LANES


@functools.partial(
    pl.kernel,
    out_type=jax.ShapeDtypeStruct((NC, NPAD, DEGW), jnp.float32),
    mesh=_sc_mesh(),
    compiler_params=pltpu.CompilerParams(needs_layout_passes=False),
    scratch_types=[
        pltpu.VMEM((EPW,), jnp.int32),
        pltpu.VMEM((NPAD,), jnp.float32),
        pltpu.VMEM((NS, RPT), jnp.float32),
        pltpu.VMEM((RPT, DEGW), jnp.float32),
        pltpu.VMEM_SHARED((NS, NS, RPT), jnp.float32),
    ],
)
def _deg_kernel(dst_hbm, out_hbm, dst_v, hist, part_v, deg_v, shared):
    cid = lax.axis_index("c")
    sid = lax.axis_index("s")
    wid = cid * NS + sid
    zero = jnp.zeros((LANES,), jnp.float32)

    def zloop(k, carry):
        hist[pl.ds(k * LANES, LANES)] = zero
        return carry

    lax.fori_loop(0, NPAD // LANES, zloop, 0)
    off = pl.multiple_of(wid * EPW, 8)
    pltpu.sync_copy(dst_hbm.at[pl.ds(off, EPW)], dst_v)

    def hloop(g, carry):
        d = dst_v[pl.ds(g * LANES, LANES)]
        occ, last = plsc.scan_count(d)
        plsc.addupdate_scatter(
            hist, (d,), lax.convert_element_type(occ, jnp.float32), mask=last
        )
        return carry

    lax.fori_loop(0, NGRP, hloop, 0)

    for t in range(NS):
        pltpu.sync_copy(hist.at[pl.ds(t * RPT, RPT)], shared.at[t, sid])
    plsc.subcore_barrier()
    pltpu.sync_copy(shared.at[sid], part_v)
    for k in range(KGRP):
        acc = jnp.zeros((LANES,), jnp.float32)
        for r in range(NS):
            acc = acc + part_v[r, pl.ds(k * LANES, LANES)]
        for i in range(LANES):
            deg_v[k * LANES + i, :] = jnp.take(
                acc, jnp.full((DEGW,), i, jnp.int32)
            )
    row0 = sid * RPT
    pltpu.sync_copy(deg_v, out_hbm.at[cid, pl.ds(row0, RPT)])


D = 128


NR = 6
NI = 12


@functools.partial(
    pl.kernel,
    out_type=jax.ShapeDtypeStruct((NC, NPAD, D), jnp.float32),
    mesh=_sc_mesh(),
    scratch_types=[
        pltpu.VMEM((NI, C), jnp.int32),
        pltpu.VMEM((NI, C), jnp.int32),
        pltpu.VMEM((NR, C, D), jnp.float32),
        pltpu.VMEM((8, D), jnp.float32),
        pltpu.VMEM_SHARED((NPAD, D), jnp.float32),
    ]
    + [pltpu.SemaphoreType.DMA] * (2 * NR + NI),
)
def _scatter(hp_hbm, src_hbm, dst_hbm, out_hbm, src_v, dst_v, rows_v, zb, acc,
             *sems):
    cid = lax.axis_index("c")
    sid = lax.axis_index("s")
    wid = cid * NS + sid
    s_g = sems[:NR]
    s_s = sems[NR:2 * NR]
    s_i = sems[2 * NR:]
    zero = jnp.zeros((LANES,), jnp.float32)
    for r in range(8):
        for l in range(D // LANES):
            zb[r, pl.ds(l * LANES, LANES)] = zero
    row0 = sid * RPT

    def zloop(k, carry):
        pltpu.sync_copy(zb, acc.at[pl.ds(row0 + k * 8, 8)])
        return carry

    lax.fori_loop(0, RPT // 8, zloop, 0)

    def fetch_idx(j, q):
        off = pl.multiple_of(wid * EPW + j * C, 8)
        pltpu.async_copy(src_hbm.at[pl.ds(off, C)], src_v.at[q], s_i[q])
        pltpu.async_copy(dst_hbm.at[pl.ds(off, C)], dst_v.at[q], s_i[q])

    def wait_idx(q):
        pltpu.make_async_copy(src_hbm.at[pl.ds(0, C)], src_v.at[q], s_i[q]).wait()
        pltpu.make_async_copy(dst_hbm.at[pl.ds(0, C)], dst_v.at[q], s_i[q]).wait()

    def start_gather(b, q):
        pltpu.async_copy(hp_hbm.at[src_v.at[q]], rows_v.at[b], s_g[b])

    def wait_gather(b, q):
        pltpu.make_async_copy(hp_hbm.at[src_v.at[q]], rows_v.at[b], s_g[b]).wait()

    def ascatter(b, q):
        pltpu.async_copy(rows_v.at[b], acc.at[dst_v.at[q]], s_s[b], add=True)

    def wscatter(b, q):
        pltpu.make_async_copy(rows_v.at[b], acc.at[dst_v.at[q]], s_s[b]).wait()

    plsc.subcore_barrier()

    def step(j, b, q, retire, fetch_ok, gather_ok):
        wait_gather(b, q)
        ascatter(b, q)
        if retire:
            wscatter((b + 3) % NR, (q + 9) % NI)
        if fetch_ok:
            fetch_idx(j + 5, (q + 5) % NI)
        if gather_ok:
            wait_idx((q + 3) % NI)
            start_gather((b + 3) % NR, (q + 3) % NI)

    for q in range(5):
        fetch_idx(q, q)
    for b in range(3):
        wait_idx(b)
        start_gather(b, b)
    for j in (0, 1, 2):
        step(j, j % NR, j % NI, False, True, True)

    UNROLL = 12

    def cloop(g, carry):
        j0 = g * UNROLL + 3
        for k in range(UNROLL):
            step(j0 + k, (3 + k) % NR, (3 + k) % NI, True, True, True)
        return carry

    nloop = (NCH - 3 - 7) // UNROLL
    lax.fori_loop(0, nloop, cloop, 0)
    for j in range(3 + nloop * UNROLL, NCH):
        step(j, j % NR, j % NI, True, j + 5 < NCH, j + 3 < NCH)
    for j in range(NCH - 3, NCH):
        wscatter(j % NR, j % NI)

    plsc.subcore_barrier()
    pltpu.sync_copy(acc.at[pl.ds(row0, RPT)], out_hbm.at[cid, pl.ds(row0, RPT)])


def _rows(i):
    return lax.broadcasted_iota(jnp.int32, (BN, 1), 0) + i * BN


def _tc_matmul1(x_pad, W1):
    def body(x_ref, w_ref, h_ref):
        h_ref[...] = jnp.dot(
            x_ref[...], w_ref[...], preferred_element_type=jnp.float32
        )

    return pl.pallas_call(
        body,
        grid=(GRID,),
        in_specs=[
            pl.BlockSpec((BN, IN_CH), lambda i: (i, 0)),
            pl.BlockSpec((IN_CH, HID_CH), lambda i: (0, 0)),
        ],
        out_specs=pl.BlockSpec((BN, HID_CH), lambda i: (i, 0)),
        out_shape=jax.ShapeDtypeStruct((NPAD, HID_CH), jnp.float32),
    )(x_pad, W1)


def _tc_scale(h1, degp):
    def body(h_ref, degp_ref, hp_ref, dis_ref):
        deg = degp_ref[0, :, 0:1] + degp_ref[1, :, 0:1] + 1.0
        dis = lax.rsqrt(deg)
        hp = jnp.where(_rows(pl.program_id(0)) < N, h_ref[...] * dis, 0.0)
        hp_ref[...] = hp
        dis_ref[...] = jnp.broadcast_to(dis, (BN, LANES))

    return pl.pallas_call(
        body,
        grid=(GRID,),
        in_specs=[
            pl.BlockSpec((BN, HID_CH), lambda i: (i, 0)),
            pl.BlockSpec((NC, BN, DEGW), lambda i: (0, i, 0)),
        ],
        out_specs=[
            pl.BlockSpec((BN, HID_CH), lambda i: (i, 0)),
            pl.BlockSpec((BN, LANES), lambda i: (i, 0)),
        ],
        out_shape=[
            jax.ShapeDtypeStruct((NPAD, HID_CH), jnp.float32),
            jax.ShapeDtypeStruct((NPAD, LANES), jnp.float32),
        ],
    )(h1, degp)


def _tc_mid(hp1, agg1, dis, b1, W2):
    def body(hp1_ref, agg_ref, dis_ref, b_ref, w_ref, hp2_ref):
        dis_c = dis_ref[:, 0:1]
        s = agg_ref[0] + agg_ref[1] + hp1_ref[...]
        x2 = jnp.maximum(s * dis_c + b_ref[...], 0.0)
        h2 = jnp.dot(x2, w_ref[...], preferred_element_type=jnp.float32)
        hp2 = jnp.where(_rows(pl.program_id(0)) < N, h2 * dis_c, 0.0)
        hp2_ref[...] = jnp.concatenate(
            [hp2, jnp.zeros((BN, HID_CH - OUT_CH), jnp.float32)], axis=1
        )

    return pl.pallas_call(
        body,
        grid=(GRID,),
        in_specs=[
            pl.BlockSpec((BN, HID_CH), lambda i: (i, 0)),
            pl.BlockSpec((NC, BN, HID_CH), lambda i: (0, i, 0)),
            pl.BlockSpec((BN, LANES), lambda i: (i, 0)),
            pl.BlockSpec((1, HID_CH), lambda i: (0, 0)),
            pl.BlockSpec((HID_CH, OUT_CH), lambda i: (0, 0)),
        ],
        out_specs=pl.BlockSpec((BN, HID_CH), lambda i: (i, 0)),
        out_shape=jax.ShapeDtypeStruct((NPAD, HID_CH), jnp.float32),
    )(hp1, agg1, dis, b1, W2)


def _tc_last(hp2, agg2, dis, b2):
    def body(hp2_ref, agg_ref, dis_ref, b_ref, out_ref):
        dis_c = dis_ref[:, 0:1]
        s = agg_ref[0, :, :OUT_CH] + agg_ref[1, :, :OUT_CH] + hp2_ref[:, :OUT_CH]
        out_ref[...] = dis_c * s + b_ref[...]

    return pl.pallas_call(
        body,
        grid=(GRID,),
        in_specs=[
            pl.BlockSpec((BN, HID_CH), lambda i: (i, 0)),
            pl.BlockSpec((NC, BN, HID_CH), lambda i: (0, i, 0)),
            pl.BlockSpec((BN, LANES), lambda i: (i, 0)),
            pl.BlockSpec((1, OUT_CH), lambda i: (0, 0)),
        ],
        out_specs=pl.BlockSpec((BN, OUT_CH), lambda i: (i, 0)),
        out_shape=jax.ShapeDtypeStruct((NPAD, OUT_CH), jnp.float32),
    )(hp2, agg2, dis, b2)


def kernel(x, edge_index, W1, b1, W2, b2):
    src = edge_index[0].astype(jnp.int32)
    dst = edge_index[1].astype(jnp.int32)
    x_pad = jnp.pad(x, ((0, NPAD - N), (0, 0)))
    degp = _deg_kernel(dst)
    h1 = _tc_matmul1(x_pad, W1)
    hp1, dis = _tc_scale(h1, degp)
    agg1 = _scatter(hp1, src, dst)
    hp2 = _tc_mid(hp1, agg1, dis, b1.reshape(1, HID_CH), W2)
    agg2 = _scatter(hp2, src, dst)
    out = _tc_last(hp2, agg2, dis, b2.reshape(1, OUT_CH))
    return out[:N]


# speedup vs baseline: 28.6479x; 1.0550x over previous
import functools

import jax
import jax.numpy as jnp
from jax import lax
from jax.experimental import pallas as pl
from jax.experimental.pallas import tpu as pltpu
from jax.experimental.pallas import tpu_sc as plsc

N = 10000
NPAD = 10240
E = 320000
IN_CH = 128
HID_CH = 128
OUT_CH = 64
LANES = 16

NC, NS = 2, 16
NW = NC * NS
EPW = E // NW
C = 40
NCH = EPW // C
RPT = NPAD // NS

BN = 256
GRID = NPAD // BN


def _sc_mesh():
    return plsc.VectorSubcoreMesh(
        core_axis_name="c", subcore_axis_name="s", num_cores=NC, num_subcores=NS
    )


DEGW = 16
NGRP = EPW // LANES
KGRP = RPT // LANES


@functools.partial(
    pl.kernel,
    out_type=jax.ShapeDtypeStruct((NC, NPAD, DEGW), jnp.float32),
    mesh=_sc_mesh(),
    compiler_params=pltpu.CompilerParams(needs_layout_passes=False),
    scratch_types=[
        pltpu.VMEM((EPW,), jnp.int32),
        pltpu.VMEM((NPAD,), jnp.float32),
        pltpu.VMEM((NS, RPT), jnp.float32),
        pltpu.VMEM((RPT, DEGW), jnp.float32),
        pltpu.VMEM_SHARED((NS, NS, RPT), jnp.float32),
    ],
)
def _deg_kernel(dst_hbm, out_hbm, dst_v, hist, part_v, deg_v, shared):
    cid = lax.axis_index("c")
    sid = lax.axis_index("s")
    wid = cid * NS + sid
    zero = jnp.zeros((LANES,), jnp.float32)

    def zloop(k, carry):
        hist[pl.ds(k * LANES, LANES)] = zero
        return carry

    lax.fori_loop(0, NPAD // LANES, zloop, 0)
    off = pl.multiple_of(wid * EPW, 8)
    pltpu.sync_copy(dst_hbm.at[pl.ds(off, EPW)], dst_v)

    def hloop(g, carry):
        d = dst_v[pl.ds(g * LANES, LANES)]
        occ, last = plsc.scan_count(d)
        plsc.addupdate_scatter(
            hist, (d,), lax.convert_element_type(occ, jnp.float32), mask=last
        )
        return carry

    lax.fori_loop(0, NGRP, hloop, 0)

    for t in range(NS):
        pltpu.sync_copy(hist.at[pl.ds(t * RPT, RPT)], shared.at[t, sid])
    plsc.subcore_barrier()
    pltpu.sync_copy(shared.at[sid], part_v)
    for k in range(KGRP):
        acc = jnp.zeros((LANES,), jnp.float32)
        for r in range(NS):
            acc = acc + part_v[r, pl.ds(k * LANES, LANES)]
        for i in range(LANES):
            deg_v[k * LANES + i, :] = jnp.take(
                acc, jnp.full((DEGW,), i, jnp.int32)
            )
    row0 = sid * RPT
    pltpu.sync_copy(deg_v, out_hbm.at[cid, pl.ds(row0, RPT)])


D = 128


NR = 7
NI = 14


@functools.partial(
    pl.kernel,
    out_type=jax.ShapeDtypeStruct((NC, NPAD, D), jnp.float32),
    mesh=_sc_mesh(),
    scratch_types=[
        pltpu.VMEM((NI, C), jnp.int32),
        pltpu.VMEM((NI, C), jnp.int32),
        pltpu.VMEM((NR, C, D), jnp.float32),
        pltpu.VMEM((8, D), jnp.float32),
        pltpu.VMEM_SHARED((NPAD, D), jnp.float32),
    ]
    + [pltpu.SemaphoreType.DMA] * (2 * NR + NI),
)
def _scatter(hp_hbm, src_hbm, dst_hbm, out_hbm, src_v, dst_v, rows_v, zb, acc,
             *sems):
    cid = lax.axis_index("c")
    sid = lax.axis_index("s")
    wid = cid * NS + sid
    s_g = sems[:NR]
    s_s = sems[NR:2 * NR]
    s_i = sems[2 * NR:]
    zero = jnp.zeros((LANES,), jnp.float32)
    for r in range(8):
        for l in range(D // LANES):
            zb[r, pl.ds(l * LANES, LANES)] = zero
    row0 = sid * RPT

    def zloop(k, carry):
        pltpu.sync_copy(zb, acc.at[pl.ds(row0 + k * 8, 8)])
        return carry

    lax.fori_loop(0, RPT // 8, zloop, 0)

    def fetch_idx(j, q):
        off = pl.multiple_of(wid * EPW + j * C, 8)
        pltpu.async_copy(src_hbm.at[pl.ds(off, C)], src_v.at[q], s_i[q])
        pltpu.async_copy(dst_hbm.at[pl.ds(off, C)], dst_v.at[q], s_i[q])

    def wait_idx(q):
        pltpu.make_async_copy(src_hbm.at[pl.ds(0, C)], src_v.at[q], s_i[q]).wait()
        pltpu.make_async_copy(dst_hbm.at[pl.ds(0, C)], dst_v.at[q], s_i[q]).wait()

    def start_gather(b, q):
        pltpu.async_copy(hp_hbm.at[src_v.at[q]], rows_v.at[b], s_g[b])

    def wait_gather(b, q):
        pltpu.make_async_copy(hp_hbm.at[src_v.at[q]], rows_v.at[b], s_g[b]).wait()

    def ascatter(b, q):
        pltpu.async_copy(rows_v.at[b], acc.at[dst_v.at[q]], s_s[b], add=True)

    def wscatter(b, q):
        pltpu.make_async_copy(rows_v.at[b], acc.at[dst_v.at[q]], s_s[b]).wait()

    plsc.subcore_barrier()

    def step(j, b, q, retire, fetch_ok, gather_ok):
        wait_gather(b, q)
        ascatter(b, q)
        if retire:
            wscatter((b + 4) % NR, (q + 11) % NI)
        if fetch_ok:
            fetch_idx(j + 6, (q + 6) % NI)
        if gather_ok:
            wait_idx((q + 4) % NI)
            start_gather((b + 4) % NR, (q + 4) % NI)

    for q in range(6):
        fetch_idx(q, q)
    for b in range(4):
        wait_idx(b)
        start_gather(b, b)
    for j in (0, 1, 2):
        step(j, j % NR, j % NI, False, True, True)

    UNROLL = 14

    def cloop(g, carry):
        j0 = g * UNROLL + 3
        for k in range(UNROLL):
            step(j0 + k, (3 + k) % NR, (3 + k) % NI, True, True, True)
        return carry

    nloop = (NCH - 3 - 6) // UNROLL
    lax.fori_loop(0, nloop, cloop, 0)
    for j in range(3 + nloop * UNROLL, NCH):
        step(j, j % NR, j % NI, True, j + 6 < NCH, j + 4 < NCH)
    for j in range(NCH - 3, NCH):
        wscatter(j % NR, j % NI)

    plsc.subcore_barrier()
    pltpu.sync_copy(acc.at[pl.ds(row0, RPT)], out_hbm.at[cid, pl.ds(row0, RPT)])


def _rows(i):
    return lax.broadcasted_iota(jnp.int32, (BN, 1), 0) + i * BN


def _tc_matmul1(x_pad, W1):
    def body(x_ref, w_ref, h_ref):
        h_ref[...] = jnp.dot(
            x_ref[...], w_ref[...], preferred_element_type=jnp.float32
        )

    return pl.pallas_call(
        body,
        grid=(GRID,),
        in_specs=[
            pl.BlockSpec((BN, IN_CH), lambda i: (i, 0)),
            pl.BlockSpec((IN_CH, HID_CH), lambda i: (0, 0)),
        ],
        out_specs=pl.BlockSpec((BN, HID_CH), lambda i: (i, 0)),
        out_shape=jax.ShapeDtypeStruct((NPAD, HID_CH), jnp.float32),
    )(x_pad, W1)


def _tc_scale(h1, degp):
    def body(h_ref, degp_ref, hp_ref, dis_ref):
        deg = degp_ref[0, :, 0:1] + degp_ref[1, :, 0:1] + 1.0
        dis = lax.rsqrt(deg)
        hp = jnp.where(_rows(pl.program_id(0)) < N, h_ref[...] * dis, 0.0)
        hp_ref[...] = hp
        dis_ref[...] = jnp.broadcast_to(dis, (BN, LANES))

    return pl.pallas_call(
        body,
        grid=(GRID,),
        in_specs=[
            pl.BlockSpec((BN, HID_CH), lambda i: (i, 0)),
            pl.BlockSpec((NC, BN, DEGW), lambda i: (0, i, 0)),
        ],
        out_specs=[
            pl.BlockSpec((BN, HID_CH), lambda i: (i, 0)),
            pl.BlockSpec((BN, LANES), lambda i: (i, 0)),
        ],
        out_shape=[
            jax.ShapeDtypeStruct((NPAD, HID_CH), jnp.float32),
            jax.ShapeDtypeStruct((NPAD, LANES), jnp.float32),
        ],
    )(h1, degp)


def _tc_mid(hp1, agg1, dis, b1, W2):
    def body(hp1_ref, agg_ref, dis_ref, b_ref, w_ref, hp2_ref):
        dis_c = dis_ref[:, 0:1]
        s = agg_ref[0] + agg_ref[1] + hp1_ref[...]
        x2 = jnp.maximum(s * dis_c + b_ref[...], 0.0)
        h2 = jnp.dot(x2, w_ref[...], preferred_element_type=jnp.float32)
        hp2 = jnp.where(_rows(pl.program_id(0)) < N, h2 * dis_c, 0.0)
        hp2_ref[...] = jnp.concatenate(
            [hp2, jnp.zeros((BN, HID_CH - OUT_CH), jnp.float32)], axis=1
        )

    return pl.pallas_call(
        body,
        grid=(GRID,),
        in_specs=[
            pl.BlockSpec((BN, HID_CH), lambda i: (i, 0)),
            pl.BlockSpec((NC, BN, HID_CH), lambda i: (0, i, 0)),
            pl.BlockSpec((BN, LANES), lambda i: (i, 0)),
            pl.BlockSpec((1, HID_CH), lambda i: (0, 0)),
            pl.BlockSpec((HID_CH, OUT_CH), lambda i: (0, 0)),
        ],
        out_specs=pl.BlockSpec((BN, HID_CH), lambda i: (i, 0)),
        out_shape=jax.ShapeDtypeStruct((NPAD, HID_CH), jnp.float32),
    )(hp1, agg1, dis, b1, W2)


def _tc_last(hp2, agg2, dis, b2):
    def body(hp2_ref, agg_ref, dis_ref, b_ref, out_ref):
        dis_c = dis_ref[:, 0:1]
        s = agg_ref[0, :, :OUT_CH] + agg_ref[1, :, :OUT_CH] + hp2_ref[:, :OUT_CH]
        out_ref[...] = dis_c * s + b_ref[...]

    return pl.pallas_call(
        body,
        grid=(GRID,),
        in_specs=[
            pl.BlockSpec((BN, HID_CH), lambda i: (i, 0)),
            pl.BlockSpec((NC, BN, HID_CH), lambda i: (0, i, 0)),
            pl.BlockSpec((BN, LANES), lambda i: (i, 0)),
            pl.BlockSpec((1, OUT_CH), lambda i: (0, 0)),
        ],
        out_specs=pl.BlockSpec((BN, OUT_CH), lambda i: (i, 0)),
        out_shape=jax.ShapeDtypeStruct((NPAD, OUT_CH), jnp.float32),
    )(hp2, agg2, dis, b2)


def kernel(x, edge_index, W1, b1, W2, b2):
    src = edge_index[0].astype(jnp.int32)
    dst = edge_index[1].astype(jnp.int32)
    x_pad = jnp.pad(x, ((0, NPAD - N), (0, 0)))
    degp = _deg_kernel(dst)
    h1 = _tc_matmul1(x_pad, W1)
    hp1, dis = _tc_scale(h1, degp)
    agg1 = _scatter(hp1, src, dst)
    hp2 = _tc_mid(hp1, agg1, dis, b1.reshape(1, HID_CH), W2)
    agg2 = _scatter(hp2, src, dst)
    out = _tc_last(hp2, agg2, dis, b2.reshape(1, OUT_CH))
    return out[:N]


# speedup vs baseline: 28.9649x; 1.0111x over previous
import functools

import jax
import jax.numpy as jnp
from jax import lax
from jax.experimental import pallas as pl
from jax.experimental.pallas import tpu as pltpu
from jax.experimental.pallas import tpu_sc as plsc

N = 10000
NPAD = 10240
E = 320000
IN_CH = 128
HID_CH = 128
OUT_CH = 64
LANES = 16

NC, NS = 2, 16
NW = NC * NS
EPW = E // NW
C = 40
NCH = EPW // C
RPT = NPAD // NS

BN = 256
GRID = NPAD // BN


def _sc_mesh():
    return plsc.VectorSubcoreMesh(
        core_axis_name="c", subcore_axis_name="s", num_cores=NC, num_subcores=NS
    )


DEGW = 16
NGRP = EPW // LANES
KGRP = RPT // LANES


@functools.partial(
    pl.kernel,
    out_type=jax.ShapeDtypeStruct((NC, NPAD, DEGW), jnp.float32),
    mesh=_sc_mesh(),
    compiler_params=pltpu.CompilerParams(needs_layout_passes=False),
    scratch_types=[
        pltpu.VMEM((EPW,), jnp.int32),
        pltpu.VMEM((NPAD,), jnp.float32),
        pltpu.VMEM((NS, RPT), jnp.float32),
        pltpu.VMEM((RPT, DEGW), jnp.float32),
        pltpu.VMEM_SHARED((NS, NS, RPT), jnp.float32),
    ],
)
def _deg_kernel(dst_hbm, out_hbm, dst_v, hist, part_v, deg_v, shared):
    cid = lax.axis_index("c")
    sid = lax.axis_index("s")
    wid = cid * NS + sid
    zero = jnp.zeros((LANES,), jnp.float32)

    def zloop(k, carry):
        hist[pl.ds(k * LANES, LANES)] = zero
        return carry

    lax.fori_loop(0, NPAD // LANES, zloop, 0)
    off = pl.multiple_of(wid * EPW, 8)
    pltpu.sync_copy(dst_hbm.at[pl.ds(off, EPW)], dst_v)

    def hloop(g, carry):
        d = dst_v[pl.ds(g * LANES, LANES)]
        occ, last = plsc.scan_count(d)
        plsc.addupdate_scatter(
            hist, (d,), lax.convert_element_type(occ, jnp.float32), mask=last
        )
        return carry

    lax.fori_loop(0, NGRP, hloop, 0)

    for t in range(NS):
        pltpu.sync_copy(hist.at[pl.ds(t * RPT, RPT)], shared.at[t, sid])
    plsc.subcore_barrier()
    pltpu.sync_copy(shared.at[sid], part_v)
    for k in range(KGRP):
        acc = jnp.zeros((LANES,), jnp.float32)
        for r in range(NS):
            acc = acc + part_v[r, pl.ds(k * LANES, LANES)]
        for i in range(LANES):
            deg_v[k * LANES + i, :] = jnp.take(
                acc, jnp.full((DEGW,), i, jnp.int32)
            )
    row0 = sid * RPT
    pltpu.sync_copy(deg_v, out_hbm.at[cid, pl.ds(row0, RPT)])


D = 128


NR = 7
NI = 14


@functools.partial(
    pl.kernel,
    out_type=jax.ShapeDtypeStruct((NC, NPAD, D), jnp.float32),
    mesh=_sc_mesh(),
    scratch_types=[
        pltpu.VMEM((NI, C), jnp.int32),
        pltpu.VMEM((NI, C), jnp.int32),
        pltpu.VMEM((NR, C, D), jnp.float32),
        pltpu.VMEM((8, D), jnp.float32),
        pltpu.VMEM_SHARED((NPAD, D), jnp.float32),
    ]
    + [pltpu.SemaphoreType.DMA] * (2 * NR + NI),
)
def _scatter(hp_hbm, src_hbm, dst_hbm, out_hbm, src_v, dst_v, rows_v, zb, acc,
             *sems):
    cid = lax.axis_index("c")
    sid = lax.axis_index("s")
    wid = cid * NS + sid
    s_g = sems[:NR]
    s_s = sems[NR:2 * NR]
    s_i = sems[2 * NR:]
    zero = jnp.zeros((LANES,), jnp.float32)
    for r in range(8):
        for l in range(D // LANES):
            zb[r, pl.ds(l * LANES, LANES)] = zero
    row0 = sid * RPT

    def zloop(k, carry):
        pltpu.sync_copy(zb, acc.at[pl.ds(row0 + k * 8, 8)])
        return carry

    lax.fori_loop(0, RPT // 8, zloop, 0)

    def fetch_idx(j, q):
        off = pl.multiple_of(wid * EPW + j * C, 8)
        pltpu.async_copy(src_hbm.at[pl.ds(off, C)], src_v.at[q], s_i[q])
        pltpu.async_copy(dst_hbm.at[pl.ds(off, C)], dst_v.at[q], s_i[q])

    def wait_idx(q):
        pltpu.make_async_copy(src_hbm.at[pl.ds(0, C)], src_v.at[q], s_i[q]).wait()
        pltpu.make_async_copy(dst_hbm.at[pl.ds(0, C)], dst_v.at[q], s_i[q]).wait()

    def start_gather(b, q):
        pltpu.async_copy(hp_hbm.at[src_v.at[q]], rows_v.at[b], s_g[b])

    def wait_gather(b, q):
        pltpu.make_async_copy(hp_hbm.at[src_v.at[q]], rows_v.at[b], s_g[b]).wait()

    def ascatter(b, q):
        pltpu.async_copy(rows_v.at[b], acc.at[dst_v.at[q]], s_s[b], add=True)

    def wscatter(b, q):
        pltpu.make_async_copy(rows_v.at[b], acc.at[dst_v.at[q]], s_s[b]).wait()

    plsc.subcore_barrier()

    def step(j, b, q, retire, fetch_ok, gather_ok):
        wait_gather(b, q)
        ascatter(b, q)
        if retire:
            wscatter((b + 5) % NR, (q + 12) % NI)
        if fetch_ok:
            fetch_idx(j + 7, (q + 7) % NI)
        if gather_ok:
            wait_idx((q + 5) % NI)
            start_gather((b + 5) % NR, (q + 5) % NI)

    for q in range(7):
        fetch_idx(q, q)
    for b in range(5):
        wait_idx(b)
        start_gather(b, b)
    for j in (0, 1):
        step(j, j % NR, j % NI, False, True, True)

    UNROLL = 14

    def cloop(g, carry):
        j0 = g * UNROLL + 2
        for k in range(UNROLL):
            step(j0 + k, (2 + k) % NR, (2 + k) % NI, True, True, True)
        return carry

    nloop = (NCH - 2 - 7) // UNROLL
    lax.fori_loop(0, nloop, cloop, 0)
    for j in range(2 + nloop * UNROLL, NCH):
        step(j, j % NR, j % NI, True, j + 7 < NCH, j + 5 < NCH)
    for j in range(NCH - 2, NCH):
        wscatter(j % NR, j % NI)

    plsc.subcore_barrier()
    pltpu.sync_copy(acc.at[pl.ds(row0, RPT)], out_hbm.at[cid, pl.ds(row0, RPT)])


def _rows(i):
    return lax.broadcasted_iota(jnp.int32, (BN, 1), 0) + i * BN


def _tc_matmul1(x_pad, W1):
    def body(x_ref, w_ref, h_ref):
        h_ref[...] = jnp.dot(
            x_ref[...], w_ref[...], preferred_element_type=jnp.float32
        )

    return pl.pallas_call(
        body,
        grid=(GRID,),
        in_specs=[
            pl.BlockSpec((BN, IN_CH), lambda i: (i, 0)),
            pl.BlockSpec((IN_CH, HID_CH), lambda i: (0, 0)),
        ],
        out_specs=pl.BlockSpec((BN, HID_CH), lambda i: (i, 0)),
        out_shape=jax.ShapeDtypeStruct((NPAD, HID_CH), jnp.float32),
    )(x_pad, W1)


def _tc_scale(h1, degp):
    def body(h_ref, degp_ref, hp_ref, dis_ref):
        deg = degp_ref[0, :, 0:1] + degp_ref[1, :, 0:1] + 1.0
        dis = lax.rsqrt(deg)
        hp = jnp.where(_rows(pl.program_id(0)) < N, h_ref[...] * dis, 0.0)
        hp_ref[...] = hp
        dis_ref[...] = jnp.broadcast_to(dis, (BN, LANES))

    return pl.pallas_call(
        body,
        grid=(GRID,),
        in_specs=[
            pl.BlockSpec((BN, HID_CH), lambda i: (i, 0)),
            pl.BlockSpec((NC, BN, DEGW), lambda i: (0, i, 0)),
        ],
        out_specs=[
            pl.BlockSpec((BN, HID_CH), lambda i: (i, 0)),
            pl.BlockSpec((BN, LANES), lambda i: (i, 0)),
        ],
        out_shape=[
            jax.ShapeDtypeStruct((NPAD, HID_CH), jnp.float32),
            jax.ShapeDtypeStruct((NPAD, LANES), jnp.float32),
        ],
    )(h1, degp)


def _tc_mid(hp1, agg1, dis, b1, W2):
    def body(hp1_ref, agg_ref, dis_ref, b_ref, w_ref, hp2_ref):
        dis_c = dis_ref[:, 0:1]
        s = agg_ref[0] + agg_ref[1] + hp1_ref[...]
        x2 = jnp.maximum(s * dis_c + b_ref[...], 0.0)
        h2 = jnp.dot(x2, w_ref[...], preferred_element_type=jnp.float32)
        hp2 = jnp.where(_rows(pl.program_id(0)) < N, h2 * dis_c, 0.0)
        hp2_ref[...] = jnp.concatenate(
            [hp2, jnp.zeros((BN, HID_CH - OUT_CH), jnp.float32)], axis=1
        )

    return pl.pallas_call(
        body,
        grid=(GRID,),
        in_specs=[
            pl.BlockSpec((BN, HID_CH), lambda i: (i, 0)),
            pl.BlockSpec((NC, BN, HID_CH), lambda i: (0, i, 0)),
            pl.BlockSpec((BN, LANES), lambda i: (i, 0)),
            pl.BlockSpec((1, HID_CH), lambda i: (0, 0)),
            pl.BlockSpec((HID_CH, OUT_CH), lambda i: (0, 0)),
        ],
        out_specs=pl.BlockSpec((BN, HID_CH), lambda i: (i, 0)),
        out_shape=jax.ShapeDtypeStruct((NPAD, HID_CH), jnp.float32),
    )(hp1, agg1, dis, b1, W2)


def _tc_last(hp2, agg2, dis, b2):
    def body(hp2_ref, agg_ref, dis_ref, b_ref, out_ref):
        dis_c = dis_ref[:, 0:1]
        s = agg_ref[0, :, :OUT_CH] + agg_ref[1, :, :OUT_CH] + hp2_ref[:, :OUT_CH]
        out_ref[...] = dis_c * s + b_ref[...]

    return pl.pallas_call(
        body,
        grid=(GRID,),
        in_specs=[
            pl.BlockSpec((BN, HID_CH), lambda i: (i, 0)),
            pl.BlockSpec((NC, BN, HID_CH), lambda i: (0, i, 0)),
            pl.BlockSpec((BN, LANES), lambda i: (i, 0)),
            pl.BlockSpec((1, OUT_CH), lambda i: (0, 0)),
        ],
        out_specs=pl.BlockSpec((BN, OUT_CH), lambda i: (i, 0)),
        out_shape=jax.ShapeDtypeStruct((NPAD, OUT_CH), jnp.float32),
    )(hp2, agg2, dis, b2)


def kernel(x, edge_index, W1, b1, W2, b2):
    src = edge_index[0].astype(jnp.int32)
    dst = edge_index[1].astype(jnp.int32)
    x_pad = jnp.pad(x, ((0, NPAD - N), (0, 0)))
    degp = _deg_kernel(dst)
    h1 = _tc_matmul1(x_pad, W1)
    hp1, dis = _tc_scale(h1, degp)
    agg1 = _scatter(hp1, src, dst)
    hp2 = _tc_mid(hp1, agg1, dis, b1.reshape(1, HID_CH), W2)
    agg2 = _scatter(hp2, src, dst)
    out = _tc_last(hp2, agg2, dis, b2.reshape(1, OUT_CH))
    return out[:N]


# speedup vs baseline: 29.7172x; 1.0260x over previous
import functools

import jax
import jax.numpy as jnp
from jax import lax
from jax.experimental import pallas as pl
from jax.experimental.pallas import tpu as pltpu
from jax.experimental.pallas import tpu_sc as plsc

N = 10000
NPAD = 10240
E = 320000
IN_CH = 128
HID_CH = 128
OUT_CH = 64
LANES = 16

NC, NS = 2, 16
NW = NC * NS
EPW = E // NW
C = 40
NCH = EPW // C
RPT = NPAD // NS

BN = 256
GRID = NPAD // BN


def _sc_mesh():
    return plsc.VectorSubcoreMesh(
        core_axis_name="c", subcore_axis_name="s", num_cores=NC, num_subcores=NS
    )


DEGW = 16
NGRP = EPW // LANES
KGRP = RPT // LANES


@functools.partial(
    pl.kernel,
    out_type=jax.ShapeDtypeStruct((NC, NPAD, DEGW), jnp.float32),
    mesh=_sc_mesh(),
    compiler_params=pltpu.CompilerParams(needs_layout_passes=False),
    scratch_types=[
        pltpu.VMEM((EPW,), jnp.int32),
        pltpu.VMEM((NPAD,), jnp.float32),
        pltpu.VMEM((NS, RPT), jnp.float32),
        pltpu.VMEM((RPT, DEGW), jnp.float32),
        pltpu.VMEM_SHARED((NS, NS, RPT), jnp.float32),
    ],
)
def _deg_kernel(dst_hbm, out_hbm, dst_v, hist, part_v, deg_v, shared):
    cid = lax.axis_index("c")
    sid = lax.axis_index("s")
    wid = cid * NS + sid
    zero = jnp.zeros((LANES,), jnp.float32)

    def zloop(k, carry):
        hist[pl.ds(k * LANES, LANES)] = zero
        return carry

    lax.fori_loop(0, NPAD // LANES, zloop, 0)
    off = pl.multiple_of(wid * EPW, 8)
    pltpu.sync_copy(dst_hbm.at[pl.ds(off, EPW)], dst_v)

    def hloop(g, carry):
        d = dst_v[pl.ds(g * LANES, LANES)]
        occ, last = plsc.scan_count(d)
        plsc.addupdate_scatter(
            hist, (d,), lax.convert_element_type(occ, jnp.float32), mask=last
        )
        return carry

    lax.fori_loop(0, NGRP, hloop, 0)

    for t in range(NS):
        pltpu.sync_copy(hist.at[pl.ds(t * RPT, RPT)], shared.at[t, sid])
    plsc.subcore_barrier()
    pltpu.sync_copy(shared.at[sid], part_v)
    for k in range(KGRP):
        acc = jnp.zeros((LANES,), jnp.float32)
        for r in range(NS):
            acc = acc + part_v[r, pl.ds(k * LANES, LANES)]
        for i in range(LANES):
            deg_v[k * LANES + i, :] = jnp.take(
                acc, jnp.full((DEGW,), i, jnp.int32)
            )
    row0 = sid * RPT
    pltpu.sync_copy(deg_v, out_hbm.at[cid, pl.ds(row0, RPT)])


D = 128


NR = 7
NI = 14


@functools.partial(
    pl.kernel,
    out_type=jax.ShapeDtypeStruct((NC, NPAD, D), jnp.float32),
    mesh=_sc_mesh(),
    scratch_types=[
        pltpu.VMEM((NI, C), jnp.int32),
        pltpu.VMEM((NI, C), jnp.int32),
        pltpu.VMEM((NR, C, D), jnp.float32),
        pltpu.VMEM((8, D), jnp.float32),
        pltpu.VMEM_SHARED((NPAD, D), jnp.float32),
    ]
    + [pltpu.SemaphoreType.DMA] * (2 * NR + NI),
)
def _scatter(hp_hbm, src_hbm, dst_hbm, out_hbm, src_v, dst_v, rows_v, zb, acc,
             *sems):
    cid = lax.axis_index("c")
    sid = lax.axis_index("s")
    wid = cid * NS + sid
    s_g = sems[:NR]
    s_s = sems[NR:2 * NR]
    s_i = sems[2 * NR:]
    zero = jnp.zeros((LANES,), jnp.float32)
    for r in range(8):
        for l in range(D // LANES):
            zb[r, pl.ds(l * LANES, LANES)] = zero
    row0 = sid * RPT

    def zloop(k, carry):
        pltpu.sync_copy(zb, acc.at[pl.ds(row0 + k * 8, 8)])
        return carry

    lax.fori_loop(0, RPT // 8, zloop, 0)

    def fetch_idx(j, q):
        off = pl.multiple_of(wid * EPW + j * C, 8)
        pltpu.async_copy(src_hbm.at[pl.ds(off, C)], src_v.at[q], s_i[q])
        pltpu.async_copy(dst_hbm.at[pl.ds(off, C)], dst_v.at[q], s_i[q])

    def wait_idx(q):
        pltpu.make_async_copy(src_hbm.at[pl.ds(0, C)], src_v.at[q], s_i[q]).wait()
        pltpu.make_async_copy(dst_hbm.at[pl.ds(0, C)], dst_v.at[q], s_i[q]).wait()

    def start_gather(b, q):
        pltpu.async_copy(hp_hbm.at[src_v.at[q]], rows_v.at[b], s_g[b])

    def wait_gather(b, q):
        pltpu.make_async_copy(hp_hbm.at[src_v.at[q]], rows_v.at[b], s_g[b]).wait()

    def ascatter(b, q):
        pltpu.async_copy(rows_v.at[b], acc.at[dst_v.at[q]], s_s[b], add=True)

    def wscatter(b, q):
        pltpu.make_async_copy(rows_v.at[b], acc.at[dst_v.at[q]], s_s[b]).wait()

    plsc.subcore_barrier()

    def step(j, b, q, retire, fetch_ok, gather_ok):
        wait_gather(b, q)
        ascatter(b, q)
        if retire:
            wscatter((b + 5) % NR, (q + 12) % NI)
        if fetch_ok:
            fetch_idx(j + 7, (q + 7) % NI)
        if gather_ok:
            wait_idx((q + 5) % NI)
            start_gather((b + 5) % NR, (q + 5) % NI)

    for q in range(7):
        fetch_idx(q, q)
    for b in range(5):
        wait_idx(b)
        start_gather(b, b)
    for j in (0, 1):
        step(j, j % NR, j % NI, False, True, True)

    UNROLL = 14

    def cloop(g, carry):
        j0 = g * UNROLL + 2
        for k in range(UNROLL):
            step(j0 + k, (2 + k) % NR, (2 + k) % NI, True, True, True)
        return carry

    nloop = (NCH - 2 - 7) // UNROLL
    lax.fori_loop(0, nloop, cloop, 0)
    for j in range(2 + nloop * UNROLL, NCH):
        step(j, j % NR, j % NI, True, j + 7 < NCH, j + 5 < NCH)
    for j in range(NCH - 2, NCH):
        wscatter(j % NR, j % NI)

    plsc.subcore_barrier()
    pltpu.sync_copy(acc.at[pl.ds(row0, RPT)], out_hbm.at[cid, pl.ds(row0, RPT)])


def _rows(i):
    return lax.broadcasted_iota(jnp.int32, (BN, 1), 0) + i * BN


def _tc_first(x_pad, W1, degp):
    def body(x_ref, w_ref, degp_ref, hp_ref, dis_ref):
        deg = degp_ref[0, :, 0:1] + degp_ref[1, :, 0:1] + 1.0
        dis = lax.rsqrt(deg)
        h = jnp.dot(x_ref[...], w_ref[...], preferred_element_type=jnp.float32)
        hp = jnp.where(_rows(pl.program_id(0)) < N, h * dis, 0.0)
        hp_ref[...] = hp
        dis_ref[...] = jnp.broadcast_to(dis, (BN, LANES))

    return pl.pallas_call(
        body,
        grid=(GRID,),
        in_specs=[
            pl.BlockSpec((BN, IN_CH), lambda i: (i, 0)),
            pl.BlockSpec((IN_CH, HID_CH), lambda i: (0, 0)),
            pl.BlockSpec((NC, BN, DEGW), lambda i: (0, i, 0)),
        ],
        out_specs=[
            pl.BlockSpec((BN, HID_CH), lambda i: (i, 0)),
            pl.BlockSpec((BN, LANES), lambda i: (i, 0)),
        ],
        out_shape=[
            jax.ShapeDtypeStruct((NPAD, HID_CH), jnp.float32),
            jax.ShapeDtypeStruct((NPAD, LANES), jnp.float32),
        ],
    )(x_pad, W1, degp)


def _tc_mid(hp1, agg1, dis, b1, W2):
    def body(hp1_ref, agg_ref, dis_ref, b_ref, w_ref, hp2_ref):
        dis_c = dis_ref[:, 0:1]
        s = agg_ref[0] + agg_ref[1] + hp1_ref[...]
        x2 = jnp.maximum(s * dis_c + b_ref[...], 0.0)
        h2 = jnp.dot(x2, w_ref[...], preferred_element_type=jnp.float32)
        hp2 = jnp.where(_rows(pl.program_id(0)) < N, h2 * dis_c, 0.0)
        hp2_ref[...] = jnp.concatenate(
            [hp2, jnp.zeros((BN, HID_CH - OUT_CH), jnp.float32)], axis=1
        )

    return pl.pallas_call(
        body,
        grid=(GRID,),
        in_specs=[
            pl.BlockSpec((BN, HID_CH), lambda i: (i, 0)),
            pl.BlockSpec((NC, BN, HID_CH), lambda i: (0, i, 0)),
            pl.BlockSpec((BN, LANES), lambda i: (i, 0)),
            pl.BlockSpec((1, HID_CH), lambda i: (0, 0)),
            pl.BlockSpec((HID_CH, OUT_CH), lambda i: (0, 0)),
        ],
        out_specs=pl.BlockSpec((BN, HID_CH), lambda i: (i, 0)),
        out_shape=jax.ShapeDtypeStruct((NPAD, HID_CH), jnp.float32),
    )(hp1, agg1, dis, b1, W2)


def _tc_last(hp2, agg2, dis, b2):
    def body(hp2_ref, agg_ref, dis_ref, b_ref, out_ref):
        dis_c = dis_ref[:, 0:1]
        s = agg_ref[0, :, :OUT_CH] + agg_ref[1, :, :OUT_CH] + hp2_ref[:, :OUT_CH]
        out_ref[...] = dis_c * s + b_ref[...]

    return pl.pallas_call(
        body,
        grid=(GRID,),
        in_specs=[
            pl.BlockSpec((BN, HID_CH), lambda i: (i, 0)),
            pl.BlockSpec((NC, BN, HID_CH), lambda i: (0, i, 0)),
            pl.BlockSpec((BN, LANES), lambda i: (i, 0)),
            pl.BlockSpec((1, OUT_CH), lambda i: (0, 0)),
        ],
        out_specs=pl.BlockSpec((BN, OUT_CH), lambda i: (i, 0)),
        out_shape=jax.ShapeDtypeStruct((NPAD, OUT_CH), jnp.float32),
    )(hp2, agg2, dis, b2)


def kernel(x, edge_index, W1, b1, W2, b2):
    src = edge_index[0].astype(jnp.int32)
    dst = edge_index[1].astype(jnp.int32)
    x_pad = jnp.pad(x, ((0, NPAD - N), (0, 0)))
    degp = _deg_kernel(dst)
    hp1, dis = _tc_first(x_pad, W1, degp)
    agg1 = _scatter(hp1, src, dst)
    hp2 = _tc_mid(hp1, agg1, dis, b1.reshape(1, HID_CH), W2)
    agg2 = _scatter(hp2, src, dst)
    out = _tc_last(hp2, agg2, dis, b2.reshape(1, OUT_CH))
    return out[:N]
